# baseline (speedup 1.0000x reference)
import functools

import jax
import jax.numpy as jnp
from jax import lax
from jax.experimental import pallas as pl
from jax.experimental.pallas import tpu as pltpu
from jax.experimental.pallas import tpu_sc as plsc

N = 10000
E = 320000
F = 128
NP = 10240
BLK = 1024
GRID = NP // BLK
EBLK = E // 128
NC = 2
NS = 16
SL = 2 * NP // NS

_MESH = dict(core_axis_name="c", subcore_axis_name="s", num_cores=NC,
             num_subcores=NS)
_SC_PARAMS = pltpu.CompilerParams(needs_layout_passes=False)


def _norms_kernel(deg_ref, x_ref, xs_ref, nrm_ref):
    deg = deg_ref[...]
    nrm = jnp.where(deg > 0.0, lax.rsqrt(jnp.where(deg > 0.0, deg, 1.0)), 0.0)
    nrm_ref[...] = nrm
    xs_ref[...] = x_ref[...] * nrm[:, 0:1]


def _layer1_kernel(p_ref, nrm_ref, w_ref, b_ref, hs_ref):
    s = (p_ref[0] + p_ref[1]) * nrm_ref[:, 1:2]
    h = jnp.maximum(jnp.dot(s, w_ref[...], preferred_element_type=jnp.float32)
                    + b_ref[...], 0.0)
    hs_ref[...] = h * nrm_ref[:, 2:3]


def _layer2_kernel(q_ref, nrm_ref, wm_ref, bm_ref, ws_ref, bs_ref, noise_ref,
                   z_ref, zb_ref):
    t = (q_ref[0] + q_ref[1]) * nrm_ref[:, 3:4]
    mean = jnp.dot(t, wm_ref[...], preferred_element_type=jnp.float32) + bm_ref[...]
    log_std = jnp.dot(t, ws_ref[...], preferred_element_type=jnp.float32) + bs_ref[...]
    z = mean + noise_ref[...] * jnp.exp(log_std)
    z_ref[...] = z
    zb_ref[...] = z.astype(jnp.bfloat16)


def _tc_norms(degs4, xp):
    return pl.pallas_call(
        _norms_kernel,
        grid=(GRID,),
        in_specs=[
            pl.BlockSpec((BLK, 4), lambda i: (i, 0)),
            pl.BlockSpec((BLK, F), lambda i: (i, 0)),
        ],
        out_specs=[
            pl.BlockSpec((BLK, F), lambda i: (i, 0)),
            pl.BlockSpec((BLK, 4), lambda i: (i, 0)),
        ],
        out_shape=[
            jax.ShapeDtypeStruct((NP, F), jnp.float32),
            jax.ShapeDtypeStruct((NP, 4), jnp.float32),
        ],
    )(degs4, xp)


def _tc_layer1(p2, nrm, W0, b0):
    return pl.pallas_call(
        _layer1_kernel,
        grid=(GRID,),
        in_specs=[
            pl.BlockSpec((2, BLK, F), lambda i: (0, i, 0)),
            pl.BlockSpec((BLK, 4), lambda i: (i, 0)),
            pl.BlockSpec((F, F), lambda i: (0, 0)),
            pl.BlockSpec((1, F), lambda i: (0, 0)),
        ],
        out_specs=pl.BlockSpec((BLK, F), lambda i: (i, 0)),
        out_shape=jax.ShapeDtypeStruct((NP, F), jnp.float32),
    )(p2, nrm, W0, b0.reshape(1, F))


def _tc_layer2(q2, nrm, Wm, bm, Ws, bs, noise):
    return pl.pallas_call(
        _layer2_kernel,
        grid=(GRID,),
        in_specs=[
            pl.BlockSpec((2, BLK, F), lambda i: (0, i, 0)),
            pl.BlockSpec((BLK, 4), lambda i: (i, 0)),
            pl.BlockSpec((F, F), lambda i: (0, 0)),
            pl.BlockSpec((1, F), lambda i: (0, 0)),
            pl.BlockSpec((F, F), lambda i: (0, 0)),
            pl.BlockSpec((1, F), lambda i: (0, 0)),
            pl.BlockSpec((BLK, F), lambda i: (i, 0)),
        ],
        out_specs=[
            pl.BlockSpec((BLK, F), lambda i: (i, 0)),
            pl.BlockSpec((BLK, F), lambda i: (i, 0)),
        ],
        out_shape=[
            jax.ShapeDtypeStruct((NP, F), jnp.float32),
            jax.ShapeDtypeStruct((NP, F), jnp.bfloat16),
        ],
    )(q2, nrm, Wm, bm.reshape(1, F), Ws, bs.reshape(1, F), noise)


_NB_DEG = -(-EBLK // NS)
_NB_AGG = -(-EBLK // (NC * NS))
_NB_DOT = _NB_DEG


def _sc_degrees_body(ei_ref, out_ref, idx_v, hist_v, acc_v, tmp_v, shared,
                     semu0, semu1, semv0, semv1):
    c = lax.axis_index("c")
    s = lax.axis_index("s")
    semu = (semu0, semu1)
    semv = (semv0, semv1)
    zeros16 = jnp.zeros((16,), jnp.float32)
    ones16 = jnp.ones((16,), jnp.float32)

    def zbody(i, _):
        hist_v[pl.ds(i * 16, 16)] = zeros16
        return 0
    lax.fori_loop(0, (2 * NP) // 16, zbody, 0)

    pltpu.sync_copy(ei_ref.at[c, 0, s], idx_v.at[0, 0])
    pltpu.sync_copy(ei_ref.at[c, 1, s], idx_v.at[0, 1])
    pltpu.async_copy(ei_ref.at[c, 0, s + NS], idx_v.at[1, 0], semu[1])
    pltpu.async_copy(ei_ref.at[c, 1, s + NS], idx_v.at[1, 1], semv[1])

    def ebody(i, _):
        for p in (0, 1):
            t = 2 * i + p
            b = s + NS * t

            @pl.when(b < EBLK)
            def _():
                if p == 0:
                    @pl.when(i >= 1)
                    def _():
                        pltpu.make_async_copy(ei_ref.at[c, 0, b],
                                              idx_v.at[p, 0], semu[p]).wait()
                        pltpu.make_async_copy(ei_ref.at[c, 1, b],
                                              idx_v.at[p, 1], semv[p]).wait()
                else:
                    pltpu.make_async_copy(ei_ref.at[c, 0, b],
                                          idx_v.at[p, 0], semu[p]).wait()
                    pltpu.make_async_copy(ei_ref.at[c, 1, b],
                                          idx_v.at[p, 1], semv[p]).wait()
                for j in range(8):
                    src16 = idx_v[p, 0, pl.ds(j * 16, 16)]
                    dst16 = idx_v[p, 1, pl.ds(j * 16, 16)]
                    plsc.addupdate_scatter(hist_v, [src16], ones16)
                    plsc.addupdate_scatter(hist_v, [dst16 + NP], ones16)

                @pl.when(b + 2 * NS < EBLK)
                def _():
                    pltpu.async_copy(ei_ref.at[c, 0, b + 2 * NS],
                                     idx_v.at[p, 0], semu[p])
                    pltpu.async_copy(ei_ref.at[c, 1, b + 2 * NS],
                                     idx_v.at[p, 1], semv[p])
        return 0
    lax.fori_loop(0, (_NB_DEG + 1) // 2, ebody, 0)

    pltpu.sync_copy(hist_v, shared.at[s])
    plsc.subcore_barrier()
    pltpu.sync_copy(shared.at[0, pl.ds(s * SL, SL)], acc_v)

    def rbody(p, _):
        pltpu.sync_copy(shared.at[p, pl.ds(s * SL, SL)], tmp_v)

        def abody(k, _):
            acc_v[pl.ds(k * 16, 16)] = (acc_v[pl.ds(k * 16, 16)]
                                        + tmp_v[pl.ds(k * 16, 16)])
            return 0
        lax.fori_loop(0, SL // 16, abody, 0)
        return 0
    lax.fori_loop(1, NS, rbody, 0)
    pltpu.sync_copy(acc_v, out_ref.at[c, pl.ds(s * SL, SL)])


def _sc_degrees(ei4):
    return pl.kernel(
        _sc_degrees_body,
        out_type=jax.ShapeDtypeStruct((2, 2 * NP), jnp.float32),
        mesh=plsc.VectorSubcoreMesh(**_MESH),
        compiler_params=_SC_PARAMS,
        scratch_types=[
            pltpu.VMEM((2, 2, 128), jnp.int32),
            pltpu.VMEM((2 * NP,), jnp.float32),
            pltpu.VMEM((SL,), jnp.float32),
            pltpu.VMEM((SL,), jnp.float32),
            pltpu.VMEM_SHARED((NS, 2 * NP), jnp.float32),
            pltpu.SemaphoreType.DMA,
            pltpu.SemaphoreType.DMA,
            pltpu.SemaphoreType.DMA,
            pltpu.SemaphoreType.DMA,
        ],
    )(ei4)


_RPT = NP // NS


def _sc_agg_body(rows_ref, ei_ref, out_ref, idx_v, rowbuf, accum,
                 semis0, semis1, semid0, semid1, semg0, semg1):
    c = lax.axis_index("c")
    s = lax.axis_index("s")
    wid = s * NC + c
    semis = (semis0, semis1)
    semid = (semid0, semid1)
    semg = (semg0, semg1)
    zeros16 = jnp.zeros((16,), jnp.float32)

    def zb(i, _):
        for j in range(8):
            rowbuf[0, i, pl.ds(j * 16, 16)] = zeros16
        return 0
    lax.fori_loop(0, 128, zb, 0)

    def za(k, _):
        pltpu.sync_copy(rowbuf.at[0],
                        accum.at[pl.ds(s * _RPT + k * 128, 128)])
        return 0
    lax.fori_loop(0, _RPT // 128, za, 0)
    plsc.subcore_barrier()

    W = NC * NS
    pltpu.sync_copy(ei_ref.at[0, wid], idx_v.at[0, 0])
    pltpu.sync_copy(ei_ref.at[1, wid], idx_v.at[0, 1])
    pltpu.async_copy(rows_ref.at[idx_v.at[0, 0]], rowbuf.at[0], semg[0])
    pltpu.async_copy(ei_ref.at[0, wid + W], idx_v.at[1, 0], semis[1])
    pltpu.async_copy(ei_ref.at[1, wid + W], idx_v.at[1, 1], semid[1])

    def ebody(i, _):
        for p in (0, 1):
            pn = 1 - p
            t = 2 * i + p
            b = wid + W * t

            @pl.when(b < EBLK)
            def _():
                @pl.when(b + W < EBLK)
                def _():
                    pltpu.make_async_copy(ei_ref.at[0, b + W],
                                          idx_v.at[pn, 0], semis[pn]).wait()
                    pltpu.make_async_copy(ei_ref.at[1, b + W],
                                          idx_v.at[pn, 1], semid[pn]).wait()
                    pltpu.async_copy(rows_ref.at[idx_v.at[pn, 0]],
                                     rowbuf.at[pn], semg[pn])
                pltpu.make_async_copy(rows_ref.at[idx_v.at[p, 0]],
                                      rowbuf.at[p], semg[p]).wait()
                pltpu.sync_copy(rowbuf.at[p], accum.at[idx_v.at[p, 1]],
                                add=True)

                @pl.when(b + 2 * W < EBLK)
                def _():
                    pltpu.async_copy(ei_ref.at[0, b + 2 * W],
                                     idx_v.at[p, 0], semis[p])
                    pltpu.async_copy(ei_ref.at[1, b + 2 * W],
                                     idx_v.at[p, 1], semid[p])
        return 0
    lax.fori_loop(0, (_NB_AGG + 1) // 2, ebody, 0)
    plsc.subcore_barrier()
    pltpu.sync_copy(accum.at[pl.ds(s * _RPT, _RPT)],
                    out_ref.at[c, pl.ds(s * _RPT, _RPT)])


def _sc_agg(rows, ei):
    return pl.kernel(
        _sc_agg_body,
        out_type=jax.ShapeDtypeStruct((2, NP, F), jnp.float32),
        mesh=plsc.VectorSubcoreMesh(**_MESH),
        compiler_params=_SC_PARAMS,
        scratch_types=[
            pltpu.VMEM((2, 2, 128), jnp.int32),
            pltpu.VMEM((2, 128, F), jnp.float32),
            pltpu.VMEM_SHARED((NP, F), jnp.float32),
            pltpu.SemaphoreType.DMA,
            pltpu.SemaphoreType.DMA,
            pltpu.SemaphoreType.DMA,
            pltpu.SemaphoreType.DMA,
            pltpu.SemaphoreType.DMA,
            pltpu.SemaphoreType.DMA,
        ],
    )(rows, ei)


def _sc_dots_body(z_ref, pe_ref, out_ref, idx_v, sidx, U, V, P, sbuf,
                  semiu0, semiu1, semiv0, semiv1, semu0, semu1, semv0, semv1,
                  semo0, semo1):
    c = lax.axis_index("c")
    s = lax.axis_index("s")
    semiu = (semiu0, semiu1)
    semiv = (semiv0, semiv1)
    semu = (semu0, semu1)
    semv = (semv0, semv1)
    semo = (semo0, semo1)
    iota16 = lax.iota(jnp.int32, 16)

    def shift_idx(p):
        for d in (0, 1):
            for q in range(8):
                sidx[p, d, pl.ds(q * 16, 16)] = (
                    idx_v[p, d, pl.ds(q * 16, 16)] >> 1)

    pltpu.sync_copy(pe_ref.at[c, 0, s], idx_v.at[0, 0])
    pltpu.sync_copy(pe_ref.at[c, 1, s], idx_v.at[0, 1])
    shift_idx(0)
    pltpu.async_copy(z_ref.at[sidx.at[0, 0]], U.at[0], semu[0])
    pltpu.async_copy(z_ref.at[sidx.at[0, 1]], V.at[0], semv[0])
    pltpu.async_copy(pe_ref.at[c, 0, s + NS], idx_v.at[1, 0], semiu[1])
    pltpu.async_copy(pe_ref.at[c, 1, s + NS], idx_v.at[1, 1], semiv[1])

    def ebody(i, _):
        for p in (0, 1):
            pn = 1 - p
            t = 2 * i + p
            b = s + NS * t

            @pl.when(b < EBLK)
            def _():
                pltpu.make_async_copy(z_ref.at[sidx.at[p, 0]], U.at[p],
                                      semu[p]).wait()
                pltpu.make_async_copy(z_ref.at[sidx.at[p, 1]], V.at[p],
                                      semv[p]).wait()

                @pl.when(b + NS < EBLK)
                def _():
                    pltpu.make_async_copy(pe_ref.at[c, 0, b + NS],
                                          idx_v.at[pn, 0], semiu[pn]).wait()
                    pltpu.make_async_copy(pe_ref.at[c, 1, b + NS],
                                          idx_v.at[pn, 1], semiv[pn]).wait()
                    shift_idx(pn)
                    pltpu.async_copy(z_ref.at[sidx.at[pn, 0]], U.at[pn],
                                     semu[pn])
                    pltpu.async_copy(z_ref.at[sidx.at[pn, 1]], V.at[pn],
                                     semv[pn])

                @pl.when(i >= 1)
                def _():
                    pltpu.make_async_copy(sbuf.at[p], out_ref.at[c, b],
                                          semo[p]).wait()

                def grp(gi, _):
                    offu = (idx_v[p, 0, pl.ds(gi * 16, 16)] & 1) * 64
                    offv = (idx_v[p, 1, pl.ds(gi * 16, 16)] & 1) * 64
                    for e in range(16):
                        j = gi * 16 + e
                        uoff = offu[e]
                        voff = offv[e]
                        acc = None
                        for k in range(4):
                            lu = plsc.bitcast(
                                U[p, j, pl.ds(uoff + k * 16, 16)],
                                jnp.bfloat16)
                            lv = plsc.bitcast(
                                V[p, j, pl.ds(voff + k * 16, 16)],
                                jnp.bfloat16)
                            au, bu = plsc.unpack(
                                lu, format=plsc.PackFormat.INTERLEAVED,
                                preferred_element_type=jnp.float32)
                            av, bv = plsc.unpack(
                                lv, format=plsc.PackFormat.INTERLEAVED,
                                preferred_element_type=jnp.float32)
                            term = au * av + bu * bv
                            acc = term if acc is None else acc + term
                        plsc.store_scatter(
                            P, [iota16, jnp.full((16,), e, jnp.int32)], acc)
                    sv = P[0, :]
                    for r in range(1, 16):
                        sv = sv + P[r, :]
                    sbuf[p, pl.ds(gi * 16, 16)] = sv
                    return 0
                lax.fori_loop(0, 8, grp, 0)
                pltpu.async_copy(sbuf.at[p], out_ref.at[c, b], semo[p])

                @pl.when(b + 2 * NS < EBLK)
                def _():
                    pltpu.async_copy(pe_ref.at[c, 0, b + 2 * NS],
                                     idx_v.at[p, 0], semiu[p])
                    pltpu.async_copy(pe_ref.at[c, 1, b + 2 * NS],
                                     idx_v.at[p, 1], semiv[p])
        return 0
    lax.fori_loop(0, (_NB_DOT + 1) // 2, ebody, 0)
    for p in (0, 1):
        pltpu.make_async_copy(sbuf.at[p], out_ref.at[c, s], semo[p]).wait()


def _sc_dots(zp, pe4):
    return pl.kernel(
        _sc_dots_body,
        out_type=jax.ShapeDtypeStruct((2, EBLK, 128), jnp.float32),
        mesh=plsc.VectorSubcoreMesh(**_MESH),
        compiler_params=_SC_PARAMS,
        scratch_types=[
            pltpu.VMEM((2, 2, 128), jnp.int32),
            pltpu.VMEM((2, 2, 128), jnp.int32),
            pltpu.VMEM((2, 128, F), jnp.float32),
            pltpu.VMEM((2, 128, F), jnp.float32),
            pltpu.VMEM((16, 16), jnp.float32),
            pltpu.VMEM((2, 128), jnp.float32),
            pltpu.SemaphoreType.DMA,
            pltpu.SemaphoreType.DMA,
            pltpu.SemaphoreType.DMA,
            pltpu.SemaphoreType.DMA,
            pltpu.SemaphoreType.DMA,
            pltpu.SemaphoreType.DMA,
            pltpu.SemaphoreType.DMA,
            pltpu.SemaphoreType.DMA,
            pltpu.SemaphoreType.DMA,
            pltpu.SemaphoreType.DMA,
        ],
    )(zp, pe4)


def kernel(x, block0_edge_index, block1_edge_index, pos_edge_index,
           neg_edge_index, W0, b0, Wm, bm, Ws, bs):
    xp = jnp.pad(x, ((0, NP - N), (0, 0)))

    ei4 = jnp.stack([block0_edge_index, block1_edge_index]).reshape(
        2, 2, EBLK, 128)
    degs2 = _sc_degrees(ei4)
    degs4 = degs2.reshape(4, NP).transpose(1, 0)

    xs, nrm = _tc_norms(degs4, xp)

    p2 = _sc_agg(xs, block0_edge_index.reshape(2, EBLK, 128))
    hs = _tc_layer1(p2, nrm, W0, b0)

    q2 = _sc_agg(hs, block1_edge_index.reshape(2, EBLK, 128))
    noise = jnp.pad(
        jax.random.normal(jax.random.key(42), (N, F), dtype=jnp.float32),
        ((0, NP - N), (0, 0)))
    zp, zb = _tc_layer2(q2, nrm, Wm, bm, Ws, bs, noise)

    zq = lax.bitcast_convert_type(zb.reshape(NP // 2, F, 2), jnp.float32)
    pe4 = jnp.stack([pos_edge_index, neg_edge_index]).reshape(2, 2, EBLK, 128)
    sc2 = _sc_dots(zq, pe4)

    return (sc2[0].reshape(E, 1), sc2[1].reshape(E, 1), zp[:N])


# speedup vs baseline: 1.6347x; 1.6347x over previous
import functools

import jax
import jax.numpy as jnp
from jax import lax
from jax.experimental import pallas as pl
from jax.experimental.pallas import tpu as pltpu
from jax.experimental.pallas import tpu_sc as plsc

N = 10000
E = 320000
F = 128
NP = 10240
BLK = 1024
GRID = NP // BLK
EBLK = E // 128
NC = 2
NS = 16
SL = 2 * NP // NS

_MESH = dict(core_axis_name="c", subcore_axis_name="s", num_cores=NC,
             num_subcores=NS)
_SC_PARAMS = pltpu.CompilerParams(needs_layout_passes=False)


def _norms_kernel(deg_ref, x_ref, xs_ref, nrm_ref):
    deg = deg_ref[...]
    nrm = jnp.where(deg > 0.0, lax.rsqrt(jnp.where(deg > 0.0, deg, 1.0)), 0.0)
    nrm_ref[...] = nrm
    xs_ref[...] = x_ref[...] * nrm[:, 0:1]


def _layer1_kernel(p_ref, nrm_ref, w_ref, b_ref, hs_ref):
    s = (p_ref[0] + p_ref[1]) * nrm_ref[:, 1:2]
    h = jnp.maximum(jnp.dot(s, w_ref[...], preferred_element_type=jnp.float32)
                    + b_ref[...], 0.0)
    hs_ref[...] = h * nrm_ref[:, 2:3]


def _layer2_kernel(q_ref, nrm_ref, wm_ref, bm_ref, ws_ref, bs_ref, noise_ref,
                   z_ref, zq_ref):
    t = (q_ref[0] + q_ref[1]) * nrm_ref[:, 3:4]
    mean = jnp.dot(t, wm_ref[...], preferred_element_type=jnp.float32) + bm_ref[...]
    log_std = jnp.dot(t, ws_ref[...], preferred_element_type=jnp.float32) + bs_ref[...]
    z = mean + noise_ref[...] * jnp.exp(log_std)
    z_ref[...] = z
    hi = lax.bitcast_convert_type(z.astype(jnp.bfloat16),
                                  jnp.uint16).astype(jnp.uint32) << 16
    zr = pltpu.roll(z, 64, 1)
    lo = lax.bitcast_convert_type(zr.astype(jnp.bfloat16),
                                  jnp.uint16).astype(jnp.uint32)
    zq_ref[...] = lax.bitcast_convert_type(hi | lo, jnp.float32)


def _tc_norms(degs4, xp):
    return pl.pallas_call(
        _norms_kernel,
        grid=(GRID,),
        in_specs=[
            pl.BlockSpec((BLK, 4), lambda i: (i, 0)),
            pl.BlockSpec((BLK, F), lambda i: (i, 0)),
        ],
        out_specs=[
            pl.BlockSpec((BLK, F), lambda i: (i, 0)),
            pl.BlockSpec((BLK, 4), lambda i: (i, 0)),
        ],
        out_shape=[
            jax.ShapeDtypeStruct((NP, F), jnp.float32),
            jax.ShapeDtypeStruct((NP, 4), jnp.float32),
        ],
    )(degs4, xp)


def _tc_layer1(p2, nrm, W0, b0):
    return pl.pallas_call(
        _layer1_kernel,
        grid=(GRID,),
        in_specs=[
            pl.BlockSpec((2, BLK, F), lambda i: (0, i, 0)),
            pl.BlockSpec((BLK, 4), lambda i: (i, 0)),
            pl.BlockSpec((F, F), lambda i: (0, 0)),
            pl.BlockSpec((1, F), lambda i: (0, 0)),
        ],
        out_specs=pl.BlockSpec((BLK, F), lambda i: (i, 0)),
        out_shape=jax.ShapeDtypeStruct((NP, F), jnp.float32),
    )(p2, nrm, W0, b0.reshape(1, F))


def _tc_layer2(q2, nrm, Wm, bm, Ws, bs, noise):
    return pl.pallas_call(
        _layer2_kernel,
        grid=(GRID,),
        in_specs=[
            pl.BlockSpec((2, BLK, F), lambda i: (0, i, 0)),
            pl.BlockSpec((BLK, 4), lambda i: (i, 0)),
            pl.BlockSpec((F, F), lambda i: (0, 0)),
            pl.BlockSpec((1, F), lambda i: (0, 0)),
            pl.BlockSpec((F, F), lambda i: (0, 0)),
            pl.BlockSpec((1, F), lambda i: (0, 0)),
            pl.BlockSpec((BLK, F), lambda i: (i, 0)),
        ],
        out_specs=[
            pl.BlockSpec((BLK, F), lambda i: (i, 0)),
            pl.BlockSpec((BLK, F), lambda i: (i, 0)),
        ],
        out_shape=[
            jax.ShapeDtypeStruct((NP, F), jnp.float32),
            jax.ShapeDtypeStruct((NP, F), jnp.float32),
        ],
    )(q2, nrm, Wm, bm.reshape(1, F), Ws, bs.reshape(1, F), noise)


_NB_DEG = -(-EBLK // NS)
_NB_AGG = -(-EBLK // (NC * NS))
_NB_DOT = _NB_DEG


def _sc_degrees_body(ei_ref, out_ref, idx_v, hist_v, acc_v, tmp_v, shared,
                     semu0, semu1, semv0, semv1):
    c = lax.axis_index("c")
    s = lax.axis_index("s")
    semu = (semu0, semu1)
    semv = (semv0, semv1)
    zeros16 = jnp.zeros((16,), jnp.float32)
    ones16 = jnp.ones((16,), jnp.float32)

    def zbody(i, _):
        hist_v[pl.ds(i * 16, 16)] = zeros16
        return 0
    lax.fori_loop(0, (2 * NP) // 16, zbody, 0)

    pltpu.sync_copy(ei_ref.at[c, 0, s], idx_v.at[0, 0])
    pltpu.sync_copy(ei_ref.at[c, 1, s], idx_v.at[0, 1])
    pltpu.async_copy(ei_ref.at[c, 0, s + NS], idx_v.at[1, 0], semu[1])
    pltpu.async_copy(ei_ref.at[c, 1, s + NS], idx_v.at[1, 1], semv[1])

    def ebody(i, _):
        for p in (0, 1):
            t = 2 * i + p
            b = s + NS * t

            @pl.when(b < EBLK)
            def _():
                if p == 0:
                    @pl.when(i >= 1)
                    def _():
                        pltpu.make_async_copy(ei_ref.at[c, 0, b],
                                              idx_v.at[p, 0], semu[p]).wait()
                        pltpu.make_async_copy(ei_ref.at[c, 1, b],
                                              idx_v.at[p, 1], semv[p]).wait()
                else:
                    pltpu.make_async_copy(ei_ref.at[c, 0, b],
                                          idx_v.at[p, 0], semu[p]).wait()
                    pltpu.make_async_copy(ei_ref.at[c, 1, b],
                                          idx_v.at[p, 1], semv[p]).wait()
                for j in range(8):
                    src16 = idx_v[p, 0, pl.ds(j * 16, 16)]
                    dst16 = idx_v[p, 1, pl.ds(j * 16, 16)]
                    plsc.addupdate_scatter(hist_v, [src16], ones16)
                    plsc.addupdate_scatter(hist_v, [dst16 + NP], ones16)

                @pl.when(b + 2 * NS < EBLK)
                def _():
                    pltpu.async_copy(ei_ref.at[c, 0, b + 2 * NS],
                                     idx_v.at[p, 0], semu[p])
                    pltpu.async_copy(ei_ref.at[c, 1, b + 2 * NS],
                                     idx_v.at[p, 1], semv[p])
        return 0
    lax.fori_loop(0, (_NB_DEG + 1) // 2, ebody, 0)

    pltpu.sync_copy(hist_v, shared.at[s])
    plsc.subcore_barrier()
    pltpu.sync_copy(shared.at[0, pl.ds(s * SL, SL)], acc_v)

    def rbody(p, _):
        pltpu.sync_copy(shared.at[p, pl.ds(s * SL, SL)], tmp_v)

        def abody(k, _):
            acc_v[pl.ds(k * 16, 16)] = (acc_v[pl.ds(k * 16, 16)]
                                        + tmp_v[pl.ds(k * 16, 16)])
            return 0
        lax.fori_loop(0, SL // 16, abody, 0)
        return 0
    lax.fori_loop(1, NS, rbody, 0)
    pltpu.sync_copy(acc_v, out_ref.at[c, pl.ds(s * SL, SL)])


def _sc_degrees(ei4):
    return pl.kernel(
        _sc_degrees_body,
        out_type=jax.ShapeDtypeStruct((2, 2 * NP), jnp.float32),
        mesh=plsc.VectorSubcoreMesh(**_MESH),
        compiler_params=_SC_PARAMS,
        scratch_types=[
            pltpu.VMEM((2, 2, 128), jnp.int32),
            pltpu.VMEM((2 * NP,), jnp.float32),
            pltpu.VMEM((SL,), jnp.float32),
            pltpu.VMEM((SL,), jnp.float32),
            pltpu.VMEM_SHARED((NS, 2 * NP), jnp.float32),
            pltpu.SemaphoreType.DMA,
            pltpu.SemaphoreType.DMA,
            pltpu.SemaphoreType.DMA,
            pltpu.SemaphoreType.DMA,
        ],
    )(ei4)


_RPT = NP // NS


def _sc_agg_body(rows_ref, ei_ref, out_ref, idx_v, rowbuf, accum,
                 semis0, semis1, semid0, semid1, semg0, semg1):
    c = lax.axis_index("c")
    s = lax.axis_index("s")
    wid = s * NC + c
    semis = (semis0, semis1)
    semid = (semid0, semid1)
    semg = (semg0, semg1)
    zeros16 = jnp.zeros((16,), jnp.float32)

    def zb(i, _):
        for j in range(8):
            rowbuf[0, i, pl.ds(j * 16, 16)] = zeros16
        return 0
    lax.fori_loop(0, 128, zb, 0)

    def za(k, _):
        pltpu.sync_copy(rowbuf.at[0],
                        accum.at[pl.ds(s * _RPT + k * 128, 128)])
        return 0
    lax.fori_loop(0, _RPT // 128, za, 0)
    plsc.subcore_barrier()

    W = NC * NS
    pltpu.sync_copy(ei_ref.at[0, wid], idx_v.at[0, 0])
    pltpu.sync_copy(ei_ref.at[1, wid], idx_v.at[0, 1])
    pltpu.async_copy(rows_ref.at[idx_v.at[0, 0]], rowbuf.at[0], semg[0])
    pltpu.async_copy(ei_ref.at[0, wid + W], idx_v.at[1, 0], semis[1])
    pltpu.async_copy(ei_ref.at[1, wid + W], idx_v.at[1, 1], semid[1])

    def ebody(i, _):
        for p in (0, 1):
            pn = 1 - p
            t = 2 * i + p
            b = wid + W * t

            @pl.when(b < EBLK)
            def _():
                @pl.when(b + W < EBLK)
                def _():
                    pltpu.make_async_copy(ei_ref.at[0, b + W],
                                          idx_v.at[pn, 0], semis[pn]).wait()
                    pltpu.make_async_copy(ei_ref.at[1, b + W],
                                          idx_v.at[pn, 1], semid[pn]).wait()
                    pltpu.async_copy(rows_ref.at[idx_v.at[pn, 0]],
                                     rowbuf.at[pn], semg[pn])
                pltpu.make_async_copy(rows_ref.at[idx_v.at[p, 0]],
                                      rowbuf.at[p], semg[p]).wait()
                pltpu.sync_copy(rowbuf.at[p], accum.at[idx_v.at[p, 1]],
                                add=True)

                @pl.when(b + 2 * W < EBLK)
                def _():
                    pltpu.async_copy(ei_ref.at[0, b + 2 * W],
                                     idx_v.at[p, 0], semis[p])
                    pltpu.async_copy(ei_ref.at[1, b + 2 * W],
                                     idx_v.at[p, 1], semid[p])
        return 0
    lax.fori_loop(0, (_NB_AGG + 1) // 2, ebody, 0)
    plsc.subcore_barrier()
    pltpu.sync_copy(accum.at[pl.ds(s * _RPT, _RPT)],
                    out_ref.at[c, pl.ds(s * _RPT, _RPT)])


def _sc_agg(rows, ei):
    return pl.kernel(
        _sc_agg_body,
        out_type=jax.ShapeDtypeStruct((2, NP, F), jnp.float32),
        mesh=plsc.VectorSubcoreMesh(**_MESH),
        compiler_params=_SC_PARAMS,
        scratch_types=[
            pltpu.VMEM((2, 2, 128), jnp.int32),
            pltpu.VMEM((2, 128, F), jnp.float32),
            pltpu.VMEM_SHARED((NP, F), jnp.float32),
            pltpu.SemaphoreType.DMA,
            pltpu.SemaphoreType.DMA,
            pltpu.SemaphoreType.DMA,
            pltpu.SemaphoreType.DMA,
            pltpu.SemaphoreType.DMA,
            pltpu.SemaphoreType.DMA,
        ],
    )(rows, ei)


_HIMASK = jnp.uint32(0xFFFF0000)


def _sc_dots_body(z_ref, pe_ref, out_ref, idx_v, U, V, P, sbuf,
                  semiu0, semiu1, semiv0, semiv1, semu0, semu1, semv0, semv1,
                  semo0, semo1):
    c = lax.axis_index("c")
    s = lax.axis_index("s")
    semiu = (semiu0, semiu1)
    semiv = (semiv0, semiv1)
    semu = (semu0, semu1)
    semv = (semv0, semv1)
    semo = (semo0, semo1)
    iota16 = lax.iota(jnp.int32, 16)

    pltpu.sync_copy(pe_ref.at[c, 0, s], idx_v.at[0, 0])
    pltpu.sync_copy(pe_ref.at[c, 1, s], idx_v.at[0, 1])
    pltpu.async_copy(z_ref.at[idx_v.at[0, 0]], U.at[0], semu[0])
    pltpu.async_copy(z_ref.at[idx_v.at[0, 1]], V.at[0], semv[0])
    pltpu.async_copy(pe_ref.at[c, 0, s + NS], idx_v.at[1, 0], semiu[1])
    pltpu.async_copy(pe_ref.at[c, 1, s + NS], idx_v.at[1, 1], semiv[1])

    def ebody(i, _):
        for p in (0, 1):
            pn = 1 - p
            t = 2 * i + p
            b = s + NS * t

            @pl.when(b < EBLK)
            def _():
                pltpu.make_async_copy(z_ref.at[idx_v.at[p, 0]], U.at[p],
                                      semu[p]).wait()
                pltpu.make_async_copy(z_ref.at[idx_v.at[p, 1]], V.at[p],
                                      semv[p]).wait()

                @pl.when(b + 2 * NS < EBLK)
                def _():
                    pltpu.async_copy(pe_ref.at[c, 0, b + 2 * NS],
                                     idx_v.at[p, 0], semiu[p])
                    pltpu.async_copy(pe_ref.at[c, 1, b + 2 * NS],
                                     idx_v.at[p, 1], semiv[p])

                @pl.when(b + NS < EBLK)
                def _():
                    pltpu.make_async_copy(pe_ref.at[c, 0, b + NS],
                                          idx_v.at[pn, 0], semiu[pn]).wait()
                    pltpu.make_async_copy(pe_ref.at[c, 1, b + NS],
                                          idx_v.at[pn, 1], semiv[pn]).wait()
                    pltpu.async_copy(z_ref.at[idx_v.at[pn, 0]], U.at[pn],
                                     semu[pn])
                    pltpu.async_copy(z_ref.at[idx_v.at[pn, 1]], V.at[pn],
                                     semv[pn])

                @pl.when(i >= 1)
                def _():
                    pltpu.make_async_copy(sbuf.at[p], out_ref.at[c, b],
                                          semo[p]).wait()

                def grp(gi, _):
                    for e in range(16):
                        j = gi * 16 + e
                        acc = None
                        for k in range(4):
                            wu = plsc.bitcast(U[p, j, pl.ds(k * 16, 16)],
                                              jnp.uint32)
                            wv = plsc.bitcast(V[p, j, pl.ds(k * 16, 16)],
                                              jnp.uint32)
                            hu = plsc.bitcast(wu & _HIMASK, jnp.float32)
                            hv = plsc.bitcast(wv & _HIMASK, jnp.float32)
                            lu = plsc.bitcast(wu << 16, jnp.float32)
                            lv = plsc.bitcast(wv << 16, jnp.float32)
                            term = hu * hv + lu * lv
                            acc = term if acc is None else acc + term
                        plsc.store_scatter(
                            P, [iota16, jnp.full((16,), e, jnp.int32)], acc)
                    sv = P[0, :]
                    for r in range(1, 16):
                        sv = sv + P[r, :]
                    sbuf[p, pl.ds(gi * 16, 16)] = sv
                    return 0
                lax.fori_loop(0, 8, grp, 0)
                pltpu.async_copy(sbuf.at[p], out_ref.at[c, b], semo[p])
        return 0
    lax.fori_loop(0, (_NB_DOT + 1) // 2, ebody, 0)
    for p in (0, 1):
        pltpu.make_async_copy(sbuf.at[p], out_ref.at[c, s], semo[p]).wait()


def _sc_dots(zp, pe4):
    return pl.kernel(
        _sc_dots_body,
        out_type=jax.ShapeDtypeStruct((2, EBLK, 128), jnp.float32),
        mesh=plsc.VectorSubcoreMesh(**_MESH),
        compiler_params=_SC_PARAMS,
        scratch_types=[
            pltpu.VMEM((2, 2, 128), jnp.int32),
            pltpu.VMEM((2, 128, F), jnp.float32),
            pltpu.VMEM((2, 128, F), jnp.float32),
            pltpu.VMEM((16, 16), jnp.float32),
            pltpu.VMEM((2, 128), jnp.float32),
            pltpu.SemaphoreType.DMA,
            pltpu.SemaphoreType.DMA,
            pltpu.SemaphoreType.DMA,
            pltpu.SemaphoreType.DMA,
            pltpu.SemaphoreType.DMA,
            pltpu.SemaphoreType.DMA,
            pltpu.SemaphoreType.DMA,
            pltpu.SemaphoreType.DMA,
            pltpu.SemaphoreType.DMA,
            pltpu.SemaphoreType.DMA,
        ],
    )(zp, pe4)


def kernel(x, block0_edge_index, block1_edge_index, pos_edge_index,
           neg_edge_index, W0, b0, Wm, bm, Ws, bs):
    xp = jnp.pad(x, ((0, NP - N), (0, 0)))

    ei4 = jnp.stack([block0_edge_index, block1_edge_index]).reshape(
        2, 2, EBLK, 128)
    degs2 = _sc_degrees(ei4)
    degs4 = degs2.reshape(4, NP).transpose(1, 0)

    xs, nrm = _tc_norms(degs4, xp)

    p2 = _sc_agg(xs, block0_edge_index.reshape(2, EBLK, 128))
    hs = _tc_layer1(p2, nrm, W0, b0)

    q2 = _sc_agg(hs, block1_edge_index.reshape(2, EBLK, 128))
    noise = jnp.pad(
        jax.random.normal(jax.random.key(42), (N, F), dtype=jnp.float32),
        ((0, NP - N), (0, 0)))
    zp, zq = _tc_layer2(q2, nrm, Wm, bm, Ws, bs, noise)

    pe4 = jnp.stack([pos_edge_index, neg_edge_index]).reshape(2, 2, EBLK, 128)
    sc2 = _sc_dots(zq, pe4)

    return (sc2[0].reshape(E, 1), sc2[1].reshape(E, 1), zp[:N])


# speedup vs baseline: 1.9647x; 1.2018x over previous
import functools

import jax
import jax.numpy as jnp
from jax import lax
from jax.experimental import pallas as pl
from jax.experimental.pallas import tpu as pltpu
from jax.experimental.pallas import tpu_sc as plsc

N = 10000
E = 320000
F = 128
NP = 10240
BLK = 1024
GRID = NP // BLK
EBLK = E // 128
NC = 2
NS = 16
SL = 2 * NP // NS

_MESH = dict(core_axis_name="c", subcore_axis_name="s", num_cores=NC,
             num_subcores=NS)
_SC_PARAMS = pltpu.CompilerParams(needs_layout_passes=False)
_SC_PARAMS_NT = pltpu.CompilerParams(needs_layout_passes=False,
                                     use_tc_tiling_on_sc=False)


def _norms_kernel(deg_ref, x_ref, xs_ref, nrm_ref):
    deg = deg_ref[...]
    nrm = jnp.where(deg > 0.0, lax.rsqrt(jnp.where(deg > 0.0, deg, 1.0)), 0.0)
    nrm_ref[...] = nrm
    xs_ref[...] = x_ref[...] * nrm[:, 0:1]


def _layer1_kernel(p_ref, nrm_ref, w_ref, b_ref, hs_ref):
    s = (p_ref[0] + p_ref[1]) * nrm_ref[:, 1:2]
    h = jnp.maximum(jnp.dot(s, w_ref[...], preferred_element_type=jnp.float32)
                    + b_ref[...], 0.0)
    hs_ref[...] = h * nrm_ref[:, 2:3]


def _layer2_kernel(q_ref, nrm_ref, wm_ref, bm_ref, ws_ref, bs_ref, noise_ref,
                   z_ref, zq_ref):
    t = (q_ref[0] + q_ref[1]) * nrm_ref[:, 3:4]
    mean = jnp.dot(t, wm_ref[...], preferred_element_type=jnp.float32) + bm_ref[...]
    log_std = jnp.dot(t, ws_ref[...], preferred_element_type=jnp.float32) + bs_ref[...]
    z = mean + noise_ref[...] * jnp.exp(log_std)
    z_ref[...] = z
    hi = lax.bitcast_convert_type(z[:, :F // 2].astype(jnp.bfloat16),
                                  jnp.uint16).astype(jnp.uint32) << 16
    lo = lax.bitcast_convert_type(z[:, F // 2:].astype(jnp.bfloat16),
                                  jnp.uint16).astype(jnp.uint32)
    zq_ref[...] = lax.bitcast_convert_type(hi | lo, jnp.float32)


def _tc_norms(degs4, xp):
    return pl.pallas_call(
        _norms_kernel,
        grid=(GRID,),
        in_specs=[
            pl.BlockSpec((BLK, 4), lambda i: (i, 0)),
            pl.BlockSpec((BLK, F), lambda i: (i, 0)),
        ],
        out_specs=[
            pl.BlockSpec((BLK, F), lambda i: (i, 0)),
            pl.BlockSpec((BLK, 4), lambda i: (i, 0)),
        ],
        out_shape=[
            jax.ShapeDtypeStruct((NP, F), jnp.float32),
            jax.ShapeDtypeStruct((NP, 4), jnp.float32),
        ],
    )(degs4, xp)


def _tc_layer1(p2, nrm, W0, b0):
    return pl.pallas_call(
        _layer1_kernel,
        grid=(GRID,),
        in_specs=[
            pl.BlockSpec((2, BLK, F), lambda i: (0, i, 0)),
            pl.BlockSpec((BLK, 4), lambda i: (i, 0)),
            pl.BlockSpec((F, F), lambda i: (0, 0)),
            pl.BlockSpec((1, F), lambda i: (0, 0)),
        ],
        out_specs=pl.BlockSpec((BLK, F), lambda i: (i, 0)),
        out_shape=jax.ShapeDtypeStruct((NP, F), jnp.float32),
    )(p2, nrm, W0, b0.reshape(1, F))


def _tc_layer2(q2, nrm, Wm, bm, Ws, bs, noise):
    return pl.pallas_call(
        _layer2_kernel,
        grid=(GRID,),
        in_specs=[
            pl.BlockSpec((2, BLK, F), lambda i: (0, i, 0)),
            pl.BlockSpec((BLK, 4), lambda i: (i, 0)),
            pl.BlockSpec((F, F), lambda i: (0, 0)),
            pl.BlockSpec((1, F), lambda i: (0, 0)),
            pl.BlockSpec((F, F), lambda i: (0, 0)),
            pl.BlockSpec((1, F), lambda i: (0, 0)),
            pl.BlockSpec((BLK, F), lambda i: (i, 0)),
        ],
        out_specs=[
            pl.BlockSpec((BLK, F), lambda i: (i, 0)),
            pl.BlockSpec((BLK, F // 2), lambda i: (i, 0)),
        ],
        out_shape=[
            jax.ShapeDtypeStruct((NP, F), jnp.float32),
            jax.ShapeDtypeStruct((NP, F // 2), jnp.float32),
        ],
    )(q2, nrm, Wm, bm.reshape(1, F), Ws, bs.reshape(1, F), noise)


_NB_DEG = -(-EBLK // NS)
_NB_AGG = -(-EBLK // (NC * NS))
_NB_DOT = _NB_DEG


def _sc_degrees_body(ei_ref, out_ref, idx_v, hist_v, acc_v, tmp_v, shared,
                     semu0, semu1, semv0, semv1):
    c = lax.axis_index("c")
    s = lax.axis_index("s")
    semu = (semu0, semu1)
    semv = (semv0, semv1)
    zeros16 = jnp.zeros((16,), jnp.float32)
    ones16 = jnp.ones((16,), jnp.float32)

    def zbody(i, _):
        hist_v[pl.ds(i * 16, 16)] = zeros16
        return 0
    lax.fori_loop(0, (2 * NP) // 16, zbody, 0)

    pltpu.sync_copy(ei_ref.at[c, 0, s], idx_v.at[0, 0])
    pltpu.sync_copy(ei_ref.at[c, 1, s], idx_v.at[0, 1])
    pltpu.async_copy(ei_ref.at[c, 0, s + NS], idx_v.at[1, 0], semu[1])
    pltpu.async_copy(ei_ref.at[c, 1, s + NS], idx_v.at[1, 1], semv[1])

    def ebody(i, _):
        for p in (0, 1):
            t = 2 * i + p
            b = s + NS * t

            @pl.when(b < EBLK)
            def _():
                if p == 0:
                    @pl.when(i >= 1)
                    def _():
                        pltpu.make_async_copy(ei_ref.at[c, 0, b],
                                              idx_v.at[p, 0], semu[p]).wait()
                        pltpu.make_async_copy(ei_ref.at[c, 1, b],
                                              idx_v.at[p, 1], semv[p]).wait()
                else:
                    pltpu.make_async_copy(ei_ref.at[c, 0, b],
                                          idx_v.at[p, 0], semu[p]).wait()
                    pltpu.make_async_copy(ei_ref.at[c, 1, b],
                                          idx_v.at[p, 1], semv[p]).wait()
                for j in range(8):
                    src16 = idx_v[p, 0, pl.ds(j * 16, 16)]
                    dst16 = idx_v[p, 1, pl.ds(j * 16, 16)]
                    plsc.addupdate_scatter(hist_v, [src16], ones16)
                    plsc.addupdate_scatter(hist_v, [dst16 + NP], ones16)

                @pl.when(b + 2 * NS < EBLK)
                def _():
                    pltpu.async_copy(ei_ref.at[c, 0, b + 2 * NS],
                                     idx_v.at[p, 0], semu[p])
                    pltpu.async_copy(ei_ref.at[c, 1, b + 2 * NS],
                                     idx_v.at[p, 1], semv[p])
        return 0
    lax.fori_loop(0, (_NB_DEG + 1) // 2, ebody, 0)

    pltpu.sync_copy(hist_v, shared.at[s])
    plsc.subcore_barrier()
    pltpu.sync_copy(shared.at[0, pl.ds(s * SL, SL)], acc_v)

    def rbody(p, _):
        pltpu.sync_copy(shared.at[p, pl.ds(s * SL, SL)], tmp_v)

        def abody(k, _):
            acc_v[pl.ds(k * 16, 16)] = (acc_v[pl.ds(k * 16, 16)]
                                        + tmp_v[pl.ds(k * 16, 16)])
            return 0
        lax.fori_loop(0, SL // 16, abody, 0)
        return 0
    lax.fori_loop(1, NS, rbody, 0)
    pltpu.sync_copy(acc_v, out_ref.at[c, pl.ds(s * SL, SL)])


def _sc_degrees(ei4):
    return pl.kernel(
        _sc_degrees_body,
        out_type=jax.ShapeDtypeStruct((2, 2 * NP), jnp.float32),
        mesh=plsc.VectorSubcoreMesh(**_MESH),
        compiler_params=_SC_PARAMS,
        scratch_types=[
            pltpu.VMEM((2, 2, 128), jnp.int32),
            pltpu.VMEM((2 * NP,), jnp.float32),
            pltpu.VMEM((SL,), jnp.float32),
            pltpu.VMEM((SL,), jnp.float32),
            pltpu.VMEM_SHARED((NS, 2 * NP), jnp.float32),
            pltpu.SemaphoreType.DMA,
            pltpu.SemaphoreType.DMA,
            pltpu.SemaphoreType.DMA,
            pltpu.SemaphoreType.DMA,
        ],
    )(ei4)


_RPT = NP // NS


def _sc_agg_body(rows_ref, ei_ref, out_ref, idx_v, rowbuf, accum,
                 semis0, semis1, semid0, semid1, semg0, semg1):
    c = lax.axis_index("c")
    s = lax.axis_index("s")
    wid = s * NC + c
    semis = (semis0, semis1)
    semid = (semid0, semid1)
    semg = (semg0, semg1)
    zeros16 = jnp.zeros((16,), jnp.float32)

    def zb(i, _):
        for j in range(8):
            rowbuf[0, i, pl.ds(j * 16, 16)] = zeros16
        return 0
    lax.fori_loop(0, 128, zb, 0)

    def za(k, _):
        pltpu.sync_copy(rowbuf.at[0],
                        accum.at[pl.ds(s * _RPT + k * 128, 128)])
        return 0
    lax.fori_loop(0, _RPT // 128, za, 0)
    plsc.subcore_barrier()

    W = NC * NS
    pltpu.sync_copy(ei_ref.at[0, wid], idx_v.at[0, 0])
    pltpu.sync_copy(ei_ref.at[1, wid], idx_v.at[0, 1])
    pltpu.async_copy(rows_ref.at[idx_v.at[0, 0]], rowbuf.at[0], semg[0])
    pltpu.async_copy(ei_ref.at[0, wid + W], idx_v.at[1, 0], semis[1])
    pltpu.async_copy(ei_ref.at[1, wid + W], idx_v.at[1, 1], semid[1])

    def ebody(i, _):
        for p in (0, 1):
            pn = 1 - p
            t = 2 * i + p
            b = wid + W * t

            @pl.when(b < EBLK)
            def _():
                @pl.when(b + W < EBLK)
                def _():
                    pltpu.make_async_copy(ei_ref.at[0, b + W],
                                          idx_v.at[pn, 0], semis[pn]).wait()
                    pltpu.make_async_copy(ei_ref.at[1, b + W],
                                          idx_v.at[pn, 1], semid[pn]).wait()
                    pltpu.async_copy(rows_ref.at[idx_v.at[pn, 0]],
                                     rowbuf.at[pn], semg[pn])
                pltpu.make_async_copy(rows_ref.at[idx_v.at[p, 0]],
                                      rowbuf.at[p], semg[p]).wait()
                pltpu.sync_copy(rowbuf.at[p], accum.at[idx_v.at[p, 1]],
                                add=True)

                @pl.when(b + 2 * W < EBLK)
                def _():
                    pltpu.async_copy(ei_ref.at[0, b + 2 * W],
                                     idx_v.at[p, 0], semis[p])
                    pltpu.async_copy(ei_ref.at[1, b + 2 * W],
                                     idx_v.at[p, 1], semid[p])
        return 0
    lax.fori_loop(0, (_NB_AGG + 1) // 2, ebody, 0)
    plsc.subcore_barrier()
    pltpu.sync_copy(accum.at[pl.ds(s * _RPT, _RPT)],
                    out_ref.at[c, pl.ds(s * _RPT, _RPT)])


def _sc_agg(rows, ei):
    return pl.kernel(
        _sc_agg_body,
        out_type=jax.ShapeDtypeStruct((2, NP, F), jnp.float32),
        mesh=plsc.VectorSubcoreMesh(**_MESH),
        compiler_params=_SC_PARAMS,
        scratch_types=[
            pltpu.VMEM((2, 2, 128), jnp.int32),
            pltpu.VMEM((2, 128, F), jnp.float32),
            pltpu.VMEM_SHARED((NP, F), jnp.float32),
            pltpu.SemaphoreType.DMA,
            pltpu.SemaphoreType.DMA,
            pltpu.SemaphoreType.DMA,
            pltpu.SemaphoreType.DMA,
            pltpu.SemaphoreType.DMA,
            pltpu.SemaphoreType.DMA,
        ],
    )(rows, ei)


_HIMASK = jnp.uint32(0xFFFF0000)


def _sc_dots_body(z_ref, pe_ref, out_ref, idx_v, U, V, P, sbuf,
                  semiu0, semiu1, semiv0, semiv1, semu0, semu1, semv0, semv1,
                  semo0, semo1):
    c = lax.axis_index("c")
    s = lax.axis_index("s")
    semiu = (semiu0, semiu1)
    semiv = (semiv0, semiv1)
    semu = (semu0, semu1)
    semv = (semv0, semv1)
    semo = (semo0, semo1)
    iota16 = lax.iota(jnp.int32, 16)

    pltpu.sync_copy(pe_ref.at[c, 0, s], idx_v.at[0, 0])
    pltpu.sync_copy(pe_ref.at[c, 1, s], idx_v.at[0, 1])
    pltpu.async_copy(z_ref.at[idx_v.at[0, 0]], U.at[0], semu[0])
    pltpu.async_copy(z_ref.at[idx_v.at[0, 1]], V.at[0], semv[0])
    pltpu.async_copy(pe_ref.at[c, 0, s + NS], idx_v.at[1, 0], semiu[1])
    pltpu.async_copy(pe_ref.at[c, 1, s + NS], idx_v.at[1, 1], semiv[1])

    def ebody(i, _):
        for p in (0, 1):
            pn = 1 - p
            t = 2 * i + p
            b = s + NS * t

            @pl.when(b < EBLK)
            def _():
                pltpu.make_async_copy(z_ref.at[idx_v.at[p, 0]], U.at[p],
                                      semu[p]).wait()
                pltpu.make_async_copy(z_ref.at[idx_v.at[p, 1]], V.at[p],
                                      semv[p]).wait()

                @pl.when(b + 2 * NS < EBLK)
                def _():
                    pltpu.async_copy(pe_ref.at[c, 0, b + 2 * NS],
                                     idx_v.at[p, 0], semiu[p])
                    pltpu.async_copy(pe_ref.at[c, 1, b + 2 * NS],
                                     idx_v.at[p, 1], semiv[p])

                @pl.when(b + NS < EBLK)
                def _():
                    pltpu.make_async_copy(pe_ref.at[c, 0, b + NS],
                                          idx_v.at[pn, 0], semiu[pn]).wait()
                    pltpu.make_async_copy(pe_ref.at[c, 1, b + NS],
                                          idx_v.at[pn, 1], semiv[pn]).wait()
                    pltpu.async_copy(z_ref.at[idx_v.at[pn, 0]], U.at[pn],
                                     semu[pn])
                    pltpu.async_copy(z_ref.at[idx_v.at[pn, 1]], V.at[pn],
                                     semv[pn])

                @pl.when(i >= 1)
                def _():
                    pltpu.make_async_copy(sbuf.at[p], out_ref.at[c, b],
                                          semo[p]).wait()

                def grp(gi, _):
                    for e in range(16):
                        j = gi * 16 + e
                        acc = None
                        for k in range(4):
                            wu = plsc.bitcast(U[p, j, pl.ds(k * 16, 16)],
                                              jnp.uint32)
                            wv = plsc.bitcast(V[p, j, pl.ds(k * 16, 16)],
                                              jnp.uint32)
                            hu = plsc.bitcast(wu & _HIMASK, jnp.float32)
                            hv = plsc.bitcast(wv & _HIMASK, jnp.float32)
                            lu = plsc.bitcast(wu << 16, jnp.float32)
                            lv = plsc.bitcast(wv << 16, jnp.float32)
                            term = hu * hv + lu * lv
                            acc = term if acc is None else acc + term
                        plsc.store_scatter(
                            P, [iota16, jnp.full((16,), e, jnp.int32)], acc)
                    sv = P[0, :]
                    for r in range(1, 16):
                        sv = sv + P[r, :]
                    sbuf[p, pl.ds(gi * 16, 16)] = sv
                    return 0
                lax.fori_loop(0, 8, grp, 0)
                pltpu.async_copy(sbuf.at[p], out_ref.at[c, b], semo[p])
        return 0
    lax.fori_loop(0, (_NB_DOT + 1) // 2, ebody, 0)
    for p in (0, 1):
        pltpu.make_async_copy(sbuf.at[p], out_ref.at[c, s], semo[p]).wait()


def _sc_dots(zp, pe4):
    return pl.kernel(
        _sc_dots_body,
        out_type=jax.ShapeDtypeStruct((2, EBLK, 128), jnp.float32),
        mesh=plsc.VectorSubcoreMesh(**_MESH),
        compiler_params=_SC_PARAMS_NT,
        scratch_types=[
            pltpu.VMEM((2, 2, 128), jnp.int32),
            pltpu.VMEM((2, 128, F // 2), jnp.float32),
            pltpu.VMEM((2, 128, F // 2), jnp.float32),
            pltpu.VMEM((16, 16), jnp.float32),
            pltpu.VMEM((2, 128), jnp.float32),
            pltpu.SemaphoreType.DMA,
            pltpu.SemaphoreType.DMA,
            pltpu.SemaphoreType.DMA,
            pltpu.SemaphoreType.DMA,
            pltpu.SemaphoreType.DMA,
            pltpu.SemaphoreType.DMA,
            pltpu.SemaphoreType.DMA,
            pltpu.SemaphoreType.DMA,
            pltpu.SemaphoreType.DMA,
            pltpu.SemaphoreType.DMA,
        ],
    )(zp, pe4)


def kernel(x, block0_edge_index, block1_edge_index, pos_edge_index,
           neg_edge_index, W0, b0, Wm, bm, Ws, bs):
    xp = jnp.pad(x, ((0, NP - N), (0, 0)))

    ei4 = jnp.stack([block0_edge_index, block1_edge_index]).reshape(
        2, 2, EBLK, 128)
    degs2 = _sc_degrees(ei4)
    degs4 = degs2.reshape(4, NP).transpose(1, 0)

    xs, nrm = _tc_norms(degs4, xp)

    p2 = _sc_agg(xs, block0_edge_index.reshape(2, EBLK, 128))
    hs = _tc_layer1(p2, nrm, W0, b0)

    q2 = _sc_agg(hs, block1_edge_index.reshape(2, EBLK, 128))
    noise = jnp.pad(
        jax.random.normal(jax.random.key(42), (N, F), dtype=jnp.float32),
        ((0, NP - N), (0, 0)))
    zp, zq = _tc_layer2(q2, nrm, Wm, bm, Ws, bs, noise)

    pe4 = jnp.stack([pos_edge_index, neg_edge_index]).reshape(2, 2, EBLK, 128)
    sc2 = _sc_dots(zq, pe4)

    return (sc2[0].reshape(E, 1), sc2[1].reshape(E, 1), zp[:N])


# speedup vs baseline: 2.0658x; 1.0515x over previous
import functools

import jax
import jax.numpy as jnp
from jax import lax
from jax.experimental import pallas as pl
from jax.experimental.pallas import tpu as pltpu
from jax.experimental.pallas import tpu_sc as plsc

N = 10000
E = 320000
F = 128
NP = 10240
BLK = 1024
GRID = NP // BLK
EBLK = E // 128
NC = 2
NS = 16
SL = 2 * NP // NS

_MESH = dict(core_axis_name="c", subcore_axis_name="s", num_cores=NC,
             num_subcores=NS)
_SC_PARAMS = pltpu.CompilerParams(needs_layout_passes=False)
_SC_PARAMS_NT = pltpu.CompilerParams(needs_layout_passes=False,
                                     use_tc_tiling_on_sc=False)


def _norms_kernel(deg_ref, x_ref, xs_ref, nrm_ref):
    deg = deg_ref[...]
    nrm = jnp.where(deg > 0.0, lax.rsqrt(jnp.where(deg > 0.0, deg, 1.0)), 0.0)
    nrm_ref[...] = nrm
    xs_ref[...] = x_ref[...] * nrm[:, 0:1]


def _layer1_kernel(p_ref, nrm_ref, w_ref, b_ref, hs_ref):
    s = (p_ref[0] + p_ref[1]) * nrm_ref[:, 1:2]
    h = jnp.maximum(jnp.dot(s, w_ref[...], preferred_element_type=jnp.float32)
                    + b_ref[...], 0.0)
    hs_ref[...] = h * nrm_ref[:, 2:3]


def _layer2_kernel(q_ref, nrm_ref, wm_ref, bm_ref, ws_ref, bs_ref, noise_ref,
                   z_ref, zq_ref):
    t = (q_ref[0] + q_ref[1]) * nrm_ref[:, 3:4]
    mean = jnp.dot(t, wm_ref[...], preferred_element_type=jnp.float32) + bm_ref[...]
    log_std = jnp.dot(t, ws_ref[...], preferred_element_type=jnp.float32) + bs_ref[...]
    z = mean + noise_ref[...] * jnp.exp(log_std)
    z_ref[...] = z
    hi = lax.bitcast_convert_type(z[:, :F // 2].astype(jnp.bfloat16),
                                  jnp.uint16).astype(jnp.uint32) << 16
    lo = lax.bitcast_convert_type(z[:, F // 2:].astype(jnp.bfloat16),
                                  jnp.uint16).astype(jnp.uint32)
    zq_ref[...] = lax.bitcast_convert_type(hi | lo, jnp.float32)


def _tc_norms(degs4, xp):
    return pl.pallas_call(
        _norms_kernel,
        grid=(GRID,),
        in_specs=[
            pl.BlockSpec((BLK, 4), lambda i: (i, 0)),
            pl.BlockSpec((BLK, F), lambda i: (i, 0)),
        ],
        out_specs=[
            pl.BlockSpec((BLK, F), lambda i: (i, 0)),
            pl.BlockSpec((BLK, 4), lambda i: (i, 0)),
        ],
        out_shape=[
            jax.ShapeDtypeStruct((NP, F), jnp.float32),
            jax.ShapeDtypeStruct((NP, 4), jnp.float32),
        ],
    )(degs4, xp)


def _tc_layer1(p2, nrm, W0, b0):
    return pl.pallas_call(
        _layer1_kernel,
        grid=(GRID,),
        in_specs=[
            pl.BlockSpec((2, BLK, F), lambda i: (0, i, 0)),
            pl.BlockSpec((BLK, 4), lambda i: (i, 0)),
            pl.BlockSpec((F, F), lambda i: (0, 0)),
            pl.BlockSpec((1, F), lambda i: (0, 0)),
        ],
        out_specs=pl.BlockSpec((BLK, F), lambda i: (i, 0)),
        out_shape=jax.ShapeDtypeStruct((NP, F), jnp.float32),
    )(p2, nrm, W0, b0.reshape(1, F))


def _tc_layer2(q2, nrm, Wm, bm, Ws, bs, noise):
    return pl.pallas_call(
        _layer2_kernel,
        grid=(GRID,),
        in_specs=[
            pl.BlockSpec((2, BLK, F), lambda i: (0, i, 0)),
            pl.BlockSpec((BLK, 4), lambda i: (i, 0)),
            pl.BlockSpec((F, F), lambda i: (0, 0)),
            pl.BlockSpec((1, F), lambda i: (0, 0)),
            pl.BlockSpec((F, F), lambda i: (0, 0)),
            pl.BlockSpec((1, F), lambda i: (0, 0)),
            pl.BlockSpec((BLK, F), lambda i: (i, 0)),
        ],
        out_specs=[
            pl.BlockSpec((BLK, F), lambda i: (i, 0)),
            pl.BlockSpec((BLK, F // 2), lambda i: (i, 0)),
        ],
        out_shape=[
            jax.ShapeDtypeStruct((NP, F), jnp.float32),
            jax.ShapeDtypeStruct((NP, F // 2), jnp.float32),
        ],
    )(q2, nrm, Wm, bm.reshape(1, F), Ws, bs.reshape(1, F), noise)


_NB_DEG = -(-EBLK // NS)
_NB_AGG = -(-EBLK // (NC * NS))
_NB_DOT = _NB_DEG


def _sc_degrees_body(ei_ref, out_ref, idx_v, hist_v, acc_v, tmp_v, shared,
                     semu0, semu1, semv0, semv1):
    c = lax.axis_index("c")
    s = lax.axis_index("s")
    semu = (semu0, semu1)
    semv = (semv0, semv1)
    zeros16 = jnp.zeros((16,), jnp.float32)
    ones16 = jnp.ones((16,), jnp.float32)

    def zbody(i, _):
        hist_v[pl.ds(i * 16, 16)] = zeros16
        return 0
    lax.fori_loop(0, (2 * NP) // 16, zbody, 0)

    pltpu.sync_copy(ei_ref.at[c, 0, s], idx_v.at[0, 0])
    pltpu.sync_copy(ei_ref.at[c, 1, s], idx_v.at[0, 1])
    pltpu.async_copy(ei_ref.at[c, 0, s + NS], idx_v.at[1, 0], semu[1])
    pltpu.async_copy(ei_ref.at[c, 1, s + NS], idx_v.at[1, 1], semv[1])

    def ebody(i, _):
        for p in (0, 1):
            t = 2 * i + p
            b = s + NS * t

            @pl.when(b < EBLK)
            def _():
                if p == 0:
                    @pl.when(i >= 1)
                    def _():
                        pltpu.make_async_copy(ei_ref.at[c, 0, b],
                                              idx_v.at[p, 0], semu[p]).wait()
                        pltpu.make_async_copy(ei_ref.at[c, 1, b],
                                              idx_v.at[p, 1], semv[p]).wait()
                else:
                    pltpu.make_async_copy(ei_ref.at[c, 0, b],
                                          idx_v.at[p, 0], semu[p]).wait()
                    pltpu.make_async_copy(ei_ref.at[c, 1, b],
                                          idx_v.at[p, 1], semv[p]).wait()
                for j in range(8):
                    src16 = idx_v[p, 0, pl.ds(j * 16, 16)]
                    dst16 = idx_v[p, 1, pl.ds(j * 16, 16)]
                    plsc.addupdate_scatter(hist_v, [src16], ones16)
                    plsc.addupdate_scatter(hist_v, [dst16 + NP], ones16)

                @pl.when(b + 2 * NS < EBLK)
                def _():
                    pltpu.async_copy(ei_ref.at[c, 0, b + 2 * NS],
                                     idx_v.at[p, 0], semu[p])
                    pltpu.async_copy(ei_ref.at[c, 1, b + 2 * NS],
                                     idx_v.at[p, 1], semv[p])
        return 0
    lax.fori_loop(0, (_NB_DEG + 1) // 2, ebody, 0)

    pltpu.sync_copy(hist_v, shared.at[s])
    plsc.subcore_barrier()
    pltpu.sync_copy(shared.at[0, pl.ds(s * SL, SL)], acc_v)

    def rbody(p, _):
        pltpu.sync_copy(shared.at[p, pl.ds(s * SL, SL)], tmp_v)

        def abody(k, _):
            acc_v[pl.ds(k * 16, 16)] = (acc_v[pl.ds(k * 16, 16)]
                                        + tmp_v[pl.ds(k * 16, 16)])
            return 0
        lax.fori_loop(0, SL // 16, abody, 0)
        return 0
    lax.fori_loop(1, NS, rbody, 0)
    pltpu.sync_copy(acc_v, out_ref.at[c, pl.ds(s * SL, SL)])


def _sc_degrees(ei4):
    return pl.kernel(
        _sc_degrees_body,
        out_type=jax.ShapeDtypeStruct((2, 2 * NP), jnp.float32),
        mesh=plsc.VectorSubcoreMesh(**_MESH),
        compiler_params=_SC_PARAMS,
        scratch_types=[
            pltpu.VMEM((2, 2, 128), jnp.int32),
            pltpu.VMEM((2 * NP,), jnp.float32),
            pltpu.VMEM((SL,), jnp.float32),
            pltpu.VMEM((SL,), jnp.float32),
            pltpu.VMEM_SHARED((NS, 2 * NP), jnp.float32),
            pltpu.SemaphoreType.DMA,
            pltpu.SemaphoreType.DMA,
            pltpu.SemaphoreType.DMA,
            pltpu.SemaphoreType.DMA,
        ],
    )(ei4)


_RPT = NP // NS


def _sc_agg_body(rows_ref, ei_ref, out_ref, idx_v, rowbuf, accum,
                 semis0, semis1, semid0, semid1, semg0, semg1):
    c = lax.axis_index("c")
    s = lax.axis_index("s")
    wid = s * NC + c
    semis = (semis0, semis1)
    semid = (semid0, semid1)
    semg = (semg0, semg1)
    zeros16 = jnp.zeros((16,), jnp.float32)

    def zb(i, _):
        for j in range(8):
            rowbuf[0, i, pl.ds(j * 16, 16)] = zeros16
        return 0
    lax.fori_loop(0, 128, zb, 0)

    def za(k, _):
        pltpu.sync_copy(rowbuf.at[0],
                        accum.at[pl.ds(s * _RPT + k * 128, 128)])
        return 0
    lax.fori_loop(0, _RPT // 128, za, 0)
    plsc.subcore_barrier()

    W = NC * NS
    pltpu.sync_copy(ei_ref.at[0, wid], idx_v.at[0, 0])
    pltpu.sync_copy(ei_ref.at[1, wid], idx_v.at[0, 1])
    pltpu.async_copy(rows_ref.at[idx_v.at[0, 0]], rowbuf.at[0], semg[0])
    pltpu.async_copy(ei_ref.at[0, wid + W], idx_v.at[1, 0], semis[1])
    pltpu.async_copy(ei_ref.at[1, wid + W], idx_v.at[1, 1], semid[1])

    def ebody(i, _):
        for p in (0, 1):
            pn = 1 - p
            t = 2 * i + p
            b = wid + W * t

            @pl.when(b < EBLK)
            def _():
                @pl.when(b + W < EBLK)
                def _():
                    pltpu.make_async_copy(ei_ref.at[0, b + W],
                                          idx_v.at[pn, 0], semis[pn]).wait()
                    pltpu.make_async_copy(ei_ref.at[1, b + W],
                                          idx_v.at[pn, 1], semid[pn]).wait()
                    pltpu.async_copy(rows_ref.at[idx_v.at[pn, 0]],
                                     rowbuf.at[pn], semg[pn])
                pltpu.make_async_copy(rows_ref.at[idx_v.at[p, 0]],
                                      rowbuf.at[p], semg[p]).wait()
                pltpu.sync_copy(rowbuf.at[p], accum.at[idx_v.at[p, 1]],
                                add=True)

                @pl.when(b + 2 * W < EBLK)
                def _():
                    pltpu.async_copy(ei_ref.at[0, b + 2 * W],
                                     idx_v.at[p, 0], semis[p])
                    pltpu.async_copy(ei_ref.at[1, b + 2 * W],
                                     idx_v.at[p, 1], semid[p])
        return 0
    lax.fori_loop(0, (_NB_AGG + 1) // 2, ebody, 0)
    plsc.subcore_barrier()
    pltpu.sync_copy(accum.at[pl.ds(s * _RPT, _RPT)],
                    out_ref.at[c, pl.ds(s * _RPT, _RPT)])


def _sc_agg(rows, ei):
    return pl.kernel(
        _sc_agg_body,
        out_type=jax.ShapeDtypeStruct((2, NP, F), jnp.float32),
        mesh=plsc.VectorSubcoreMesh(**_MESH),
        compiler_params=_SC_PARAMS,
        scratch_types=[
            pltpu.VMEM((2, 2, 128), jnp.int32),
            pltpu.VMEM((2, 128, F), jnp.float32),
            pltpu.VMEM_SHARED((NP, F), jnp.float32),
            pltpu.SemaphoreType.DMA,
            pltpu.SemaphoreType.DMA,
            pltpu.SemaphoreType.DMA,
            pltpu.SemaphoreType.DMA,
            pltpu.SemaphoreType.DMA,
            pltpu.SemaphoreType.DMA,
        ],
    )(rows, ei)


_DSLOT = -(-EBLK // (2 * NS))


def _sc_dots_body(z_ref, pe_ref, out_ref, idx_v, U, V, P, sbuf,
                  semiu0, semiu1, semiv0, semiv1, semu0, semu1, semv0, semv1,
                  semo0, semo1):
    c = lax.axis_index("c")
    s = lax.axis_index("s")
    semiu = (semiu0, semiu1)
    semiv = (semiv0, semiv1)
    semu = (semu0, semu1)
    semv = (semv0, semv1)
    semo = (semo0, semo1)
    iota16 = lax.iota(jnp.int32, 16)

    def start_gathers(p):
        for h in (0, 1):
            pltpu.async_copy(z_ref.at[idx_v.at[p, 0, h]],
                             U.at[p, pl.ds(h * 128, 128)], semu[p])
            pltpu.async_copy(z_ref.at[idx_v.at[p, 1, h]],
                             V.at[p, pl.ds(h * 128, 128)], semv[p])

    def wait_gathers(p):
        for h in (0, 1):
            pltpu.make_async_copy(z_ref.at[idx_v.at[p, 0, h]],
                                  U.at[p, pl.ds(h * 128, 128)],
                                  semu[p]).wait()
            pltpu.make_async_copy(z_ref.at[idx_v.at[p, 1, h]],
                                  V.at[p, pl.ds(h * 128, 128)],
                                  semv[p]).wait()

    b0 = 2 * s
    pltpu.sync_copy(pe_ref.at[c, 0, pl.ds(b0, 2)], idx_v.at[0, 0])
    pltpu.sync_copy(pe_ref.at[c, 1, pl.ds(b0, 2)], idx_v.at[0, 1])
    start_gathers(0)
    pltpu.async_copy(pe_ref.at[c, 0, pl.ds(b0 + 2 * NS, 2)], idx_v.at[1, 0],
                     semiu[1])
    pltpu.async_copy(pe_ref.at[c, 1, pl.ds(b0 + 2 * NS, 2)], idx_v.at[1, 1],
                     semiv[1])

    def ebody(i, _):
        for p in (0, 1):
            pn = 1 - p
            t = 2 * i + p
            b = 2 * s + 2 * NS * t

            @pl.when(b < EBLK)
            def _():
                wait_gathers(p)

                @pl.when(b + 4 * NS < EBLK)
                def _():
                    pltpu.async_copy(pe_ref.at[c, 0, pl.ds(b + 4 * NS, 2)],
                                     idx_v.at[p, 0], semiu[p])
                    pltpu.async_copy(pe_ref.at[c, 1, pl.ds(b + 4 * NS, 2)],
                                     idx_v.at[p, 1], semiv[p])

                @pl.when(b + 2 * NS < EBLK)
                def _():
                    pltpu.make_async_copy(
                        pe_ref.at[c, 0, pl.ds(b + 2 * NS, 2)],
                        idx_v.at[pn, 0], semiu[pn]).wait()
                    pltpu.make_async_copy(
                        pe_ref.at[c, 1, pl.ds(b + 2 * NS, 2)],
                        idx_v.at[pn, 1], semiv[pn]).wait()
                    start_gathers(pn)

                @pl.when(i >= 1)
                def _():
                    pltpu.make_async_copy(sbuf.at[p],
                                          out_ref.at[c, pl.ds(b, 2)],
                                          semo[p]).wait()

                def grp(gi, _):
                    for e in range(16):
                        j = gi * 16 + e
                        acc = None
                        for k in range(4):
                            hu = U[p, j, pl.ds(k * 16, 16)]
                            hv = V[p, j, pl.ds(k * 16, 16)]
                            lu = plsc.bitcast(
                                plsc.bitcast(hu, jnp.uint32) << 16,
                                jnp.float32)
                            lv = plsc.bitcast(
                                plsc.bitcast(hv, jnp.uint32) << 16,
                                jnp.float32)
                            term = hu * hv + lu * lv
                            acc = term if acc is None else acc + term
                        plsc.store_scatter(
                            P, [iota16, jnp.full((16,), e, jnp.int32)], acc)
                    sv = P[0, :]
                    for r in range(1, 16):
                        sv = sv + P[r, :]
                    sbuf[p, gi // 8, pl.ds((gi % 8) * 16, 16)] = sv
                    return 0
                lax.fori_loop(0, 16, grp, 0)
                pltpu.async_copy(sbuf.at[p], out_ref.at[c, pl.ds(b, 2)],
                                 semo[p])
        return 0
    lax.fori_loop(0, (_DSLOT + 1) // 2, ebody, 0)
    for p in (0, 1):
        pltpu.make_async_copy(sbuf.at[p], out_ref.at[c, pl.ds(2 * s, 2)],
                              semo[p]).wait()


def _sc_dots(zp, pe4):
    return pl.kernel(
        _sc_dots_body,
        out_type=jax.ShapeDtypeStruct((2, EBLK, 128), jnp.float32),
        mesh=plsc.VectorSubcoreMesh(**_MESH),
        compiler_params=_SC_PARAMS_NT,
        scratch_types=[
            pltpu.VMEM((2, 2, 2, 128), jnp.int32),
            pltpu.VMEM((2, 256, F // 2), jnp.float32),
            pltpu.VMEM((2, 256, F // 2), jnp.float32),
            pltpu.VMEM((16, 16), jnp.float32),
            pltpu.VMEM((2, 2, 128), jnp.float32),
            pltpu.SemaphoreType.DMA,
            pltpu.SemaphoreType.DMA,
            pltpu.SemaphoreType.DMA,
            pltpu.SemaphoreType.DMA,
            pltpu.SemaphoreType.DMA,
            pltpu.SemaphoreType.DMA,
            pltpu.SemaphoreType.DMA,
            pltpu.SemaphoreType.DMA,
            pltpu.SemaphoreType.DMA,
            pltpu.SemaphoreType.DMA,
        ],
    )(zp, pe4)


def kernel(x, block0_edge_index, block1_edge_index, pos_edge_index,
           neg_edge_index, W0, b0, Wm, bm, Ws, bs):
    xp = jnp.pad(x, ((0, NP - N), (0, 0)))

    ei4 = jnp.stack([block0_edge_index, block1_edge_index]).reshape(
        2, 2, EBLK, 128)
    degs2 = _sc_degrees(ei4)
    degs4 = degs2.reshape(4, NP).transpose(1, 0)

    xs, nrm = _tc_norms(degs4, xp)

    p2 = _sc_agg(xs, block0_edge_index.reshape(2, EBLK, 128))
    hs = _tc_layer1(p2, nrm, W0, b0)

    q2 = _sc_agg(hs, block1_edge_index.reshape(2, EBLK, 128))
    noise = jnp.pad(
        jax.random.normal(jax.random.key(42), (N, F), dtype=jnp.float32),
        ((0, NP - N), (0, 0)))
    zp, zq = _tc_layer2(q2, nrm, Wm, bm, Ws, bs, noise)

    pe4 = jnp.stack([pos_edge_index, neg_edge_index]).reshape(2, 2, EBLK, 128)
    sc2 = _sc_dots(zq, pe4)

    return (sc2[0].reshape(E, 1), sc2[1].reshape(E, 1), zp[:N])


# speedup vs baseline: 2.1173x; 1.0250x over previous
import functools

import jax
import jax.numpy as jnp
from jax import lax
from jax.experimental import pallas as pl
from jax.experimental.pallas import tpu as pltpu
from jax.experimental.pallas import tpu_sc as plsc

N = 10000
E = 320000
F = 128
NP = 10240
BLK = 1024
GRID = NP // BLK
EBLK = E // 128
NC = 2
NS = 16
SL = 2 * NP // NS

_MESH = dict(core_axis_name="c", subcore_axis_name="s", num_cores=NC,
             num_subcores=NS)
_SC_PARAMS = pltpu.CompilerParams(needs_layout_passes=False)
_SC_PARAMS_NT = pltpu.CompilerParams(needs_layout_passes=False,
                                     use_tc_tiling_on_sc=False)


def _norms_kernel(deg_ref, x_ref, xs_ref, nrm_ref):
    deg = deg_ref[...]
    nrm = jnp.where(deg > 0.0, lax.rsqrt(jnp.where(deg > 0.0, deg, 1.0)), 0.0)
    nrm_ref[...] = nrm
    xs_ref[...] = x_ref[...] * nrm[:, 0:1]


def _layer1_kernel(p_ref, nrm_ref, w_ref, b_ref, hs_ref):
    s = (p_ref[0] + p_ref[1]) * nrm_ref[:, 1:2]
    h = jnp.maximum(jnp.dot(s, w_ref[...], preferred_element_type=jnp.float32)
                    + b_ref[...], 0.0)
    hs_ref[...] = h * nrm_ref[:, 2:3]


def _layer2_kernel(q_ref, nrm_ref, wm_ref, bm_ref, ws_ref, bs_ref, noise_ref,
                   z_ref, zq_ref):
    t = (q_ref[0] + q_ref[1]) * nrm_ref[:, 3:4]
    mean = jnp.dot(t, wm_ref[...], preferred_element_type=jnp.float32) + bm_ref[...]
    log_std = jnp.dot(t, ws_ref[...], preferred_element_type=jnp.float32) + bs_ref[...]
    z = mean + noise_ref[...] * jnp.exp(log_std)
    z_ref[...] = z
    hi = lax.bitcast_convert_type(z[:, :F // 2].astype(jnp.bfloat16),
                                  jnp.uint16).astype(jnp.uint32) << 16
    lo = lax.bitcast_convert_type(z[:, F // 2:].astype(jnp.bfloat16),
                                  jnp.uint16).astype(jnp.uint32)
    zq_ref[...] = lax.bitcast_convert_type(hi | lo, jnp.float32)


def _tc_norms(degs4, xp):
    return pl.pallas_call(
        _norms_kernel,
        grid=(GRID,),
        in_specs=[
            pl.BlockSpec((BLK, 4), lambda i: (i, 0)),
            pl.BlockSpec((BLK, F), lambda i: (i, 0)),
        ],
        out_specs=[
            pl.BlockSpec((BLK, F), lambda i: (i, 0)),
            pl.BlockSpec((BLK, 4), lambda i: (i, 0)),
        ],
        out_shape=[
            jax.ShapeDtypeStruct((NP, F), jnp.float32),
            jax.ShapeDtypeStruct((NP, 4), jnp.float32),
        ],
    )(degs4, xp)


def _tc_layer1(p2, nrm, W0, b0):
    return pl.pallas_call(
        _layer1_kernel,
        grid=(GRID,),
        in_specs=[
            pl.BlockSpec((2, BLK, F), lambda i: (0, i, 0)),
            pl.BlockSpec((BLK, 4), lambda i: (i, 0)),
            pl.BlockSpec((F, F), lambda i: (0, 0)),
            pl.BlockSpec((1, F), lambda i: (0, 0)),
        ],
        out_specs=pl.BlockSpec((BLK, F), lambda i: (i, 0)),
        out_shape=jax.ShapeDtypeStruct((NP, F), jnp.float32),
    )(p2, nrm, W0, b0.reshape(1, F))


def _tc_layer2(q2, nrm, Wm, bm, Ws, bs, noise):
    return pl.pallas_call(
        _layer2_kernel,
        grid=(GRID,),
        in_specs=[
            pl.BlockSpec((2, BLK, F), lambda i: (0, i, 0)),
            pl.BlockSpec((BLK, 4), lambda i: (i, 0)),
            pl.BlockSpec((F, F), lambda i: (0, 0)),
            pl.BlockSpec((1, F), lambda i: (0, 0)),
            pl.BlockSpec((F, F), lambda i: (0, 0)),
            pl.BlockSpec((1, F), lambda i: (0, 0)),
            pl.BlockSpec((BLK, F), lambda i: (i, 0)),
        ],
        out_specs=[
            pl.BlockSpec((BLK, F), lambda i: (i, 0)),
            pl.BlockSpec((BLK, F // 2), lambda i: (i, 0)),
        ],
        out_shape=[
            jax.ShapeDtypeStruct((NP, F), jnp.float32),
            jax.ShapeDtypeStruct((NP, F // 2), jnp.float32),
        ],
    )(q2, nrm, Wm, bm.reshape(1, F), Ws, bs.reshape(1, F), noise)


_NB_DEG = -(-EBLK // NS)
_NB_AGG = -(-EBLK // (NC * NS))
_NB_DOT = _NB_DEG


def _sc_degrees_body(ei_ref, out_ref, idx_v, hist_v, acc_v, tmp_v, shared,
                     semu0, semu1, semv0, semv1):
    c = lax.axis_index("c")
    s = lax.axis_index("s")
    semu = (semu0, semu1)
    semv = (semv0, semv1)
    zeros16 = jnp.zeros((16,), jnp.float32)
    ones16 = jnp.ones((16,), jnp.float32)

    def zbody(i, _):
        hist_v[pl.ds(i * 16, 16)] = zeros16
        return 0
    lax.fori_loop(0, (2 * NP) // 16, zbody, 0)

    pltpu.sync_copy(ei_ref.at[c, 0, pl.ds(2 * s, 2)], idx_v.at[0, 0])
    pltpu.sync_copy(ei_ref.at[c, 1, pl.ds(2 * s, 2)], idx_v.at[0, 1])
    pltpu.async_copy(ei_ref.at[c, 0, pl.ds(2 * s + 2 * NS, 2)],
                     idx_v.at[1, 0], semu[1])
    pltpu.async_copy(ei_ref.at[c, 1, pl.ds(2 * s + 2 * NS, 2)],
                     idx_v.at[1, 1], semv[1])

    def ebody(i, _):
        for p in (0, 1):
            t = 2 * i + p
            b = 2 * s + 2 * NS * t

            @pl.when(b < EBLK)
            def _():
                if p == 0:
                    @pl.when(i >= 1)
                    def _():
                        pltpu.make_async_copy(ei_ref.at[c, 0, pl.ds(b, 2)],
                                              idx_v.at[p, 0], semu[p]).wait()
                        pltpu.make_async_copy(ei_ref.at[c, 1, pl.ds(b, 2)],
                                              idx_v.at[p, 1], semv[p]).wait()
                else:
                    pltpu.make_async_copy(ei_ref.at[c, 0, pl.ds(b, 2)],
                                          idx_v.at[p, 0], semu[p]).wait()
                    pltpu.make_async_copy(ei_ref.at[c, 1, pl.ds(b, 2)],
                                          idx_v.at[p, 1], semv[p]).wait()
                for j in range(16):
                    src16 = idx_v[p, 0, j // 8, pl.ds((j % 8) * 16, 16)]
                    dst16 = idx_v[p, 1, j // 8, pl.ds((j % 8) * 16, 16)]
                    plsc.addupdate_scatter(hist_v, [src16], ones16)
                    plsc.addupdate_scatter(hist_v, [dst16 + NP], ones16)

                @pl.when(b + 4 * NS < EBLK)
                def _():
                    pltpu.async_copy(ei_ref.at[c, 0, pl.ds(b + 4 * NS, 2)],
                                     idx_v.at[p, 0], semu[p])
                    pltpu.async_copy(ei_ref.at[c, 1, pl.ds(b + 4 * NS, 2)],
                                     idx_v.at[p, 1], semv[p])
        return 0
    lax.fori_loop(0, (-(-EBLK // (2 * NS)) + 1) // 2, ebody, 0)

    pltpu.sync_copy(hist_v, shared.at[s])
    plsc.subcore_barrier()
    pltpu.sync_copy(shared.at[0, pl.ds(s * SL, SL)], acc_v)

    def rbody(p, _):
        pltpu.sync_copy(shared.at[p, pl.ds(s * SL, SL)], tmp_v)

        def abody(k, _):
            acc_v[pl.ds(k * 16, 16)] = (acc_v[pl.ds(k * 16, 16)]
                                        + tmp_v[pl.ds(k * 16, 16)])
            return 0
        lax.fori_loop(0, SL // 16, abody, 0)
        return 0
    lax.fori_loop(1, NS, rbody, 0)
    pltpu.sync_copy(acc_v, out_ref.at[c, pl.ds(s * SL, SL)])


def _sc_degrees(ei4):
    return pl.kernel(
        _sc_degrees_body,
        out_type=jax.ShapeDtypeStruct((2, 2 * NP), jnp.float32),
        mesh=plsc.VectorSubcoreMesh(**_MESH),
        compiler_params=_SC_PARAMS,
        scratch_types=[
            pltpu.VMEM((2, 2, 2, 128), jnp.int32),
            pltpu.VMEM((2 * NP,), jnp.float32),
            pltpu.VMEM((SL,), jnp.float32),
            pltpu.VMEM((SL,), jnp.float32),
            pltpu.VMEM_SHARED((NS, 2 * NP), jnp.float32),
            pltpu.SemaphoreType.DMA,
            pltpu.SemaphoreType.DMA,
            pltpu.SemaphoreType.DMA,
            pltpu.SemaphoreType.DMA,
        ],
    )(ei4)


_RPT = NP // NS


def _sc_agg_body(rows_ref, ei_ref, out_ref, idx_v, rowbuf, accum,
                 semis0, semis1, semid0, semid1, semg0, semg1):
    c = lax.axis_index("c")
    s = lax.axis_index("s")
    wid = s * NC + c
    semis = (semis0, semis1)
    semid = (semid0, semid1)
    semg = (semg0, semg1)
    zeros16 = jnp.zeros((16,), jnp.float32)

    def zb(i, _):
        for j in range(8):
            rowbuf[0, i, pl.ds(j * 16, 16)] = zeros16
        return 0
    lax.fori_loop(0, 128, zb, 0)

    def za(k, _):
        pltpu.sync_copy(rowbuf.at[0],
                        accum.at[pl.ds(s * _RPT + k * 128, 128)])
        return 0
    lax.fori_loop(0, _RPT // 128, za, 0)
    plsc.subcore_barrier()

    W = NC * NS
    pltpu.sync_copy(ei_ref.at[0, wid], idx_v.at[0, 0])
    pltpu.sync_copy(ei_ref.at[1, wid], idx_v.at[0, 1])
    pltpu.async_copy(rows_ref.at[idx_v.at[0, 0]], rowbuf.at[0], semg[0])
    pltpu.async_copy(ei_ref.at[0, wid + W], idx_v.at[1, 0], semis[1])
    pltpu.async_copy(ei_ref.at[1, wid + W], idx_v.at[1, 1], semid[1])

    def ebody(i, _):
        for p in (0, 1):
            pn = 1 - p
            t = 2 * i + p
            b = wid + W * t

            @pl.when(b < EBLK)
            def _():
                @pl.when(b + W < EBLK)
                def _():
                    pltpu.make_async_copy(ei_ref.at[0, b + W],
                                          idx_v.at[pn, 0], semis[pn]).wait()
                    pltpu.make_async_copy(ei_ref.at[1, b + W],
                                          idx_v.at[pn, 1], semid[pn]).wait()
                    pltpu.async_copy(rows_ref.at[idx_v.at[pn, 0]],
                                     rowbuf.at[pn], semg[pn])
                pltpu.make_async_copy(rows_ref.at[idx_v.at[p, 0]],
                                      rowbuf.at[p], semg[p]).wait()
                pltpu.sync_copy(rowbuf.at[p], accum.at[idx_v.at[p, 1]],
                                add=True)

                @pl.when(b + 2 * W < EBLK)
                def _():
                    pltpu.async_copy(ei_ref.at[0, b + 2 * W],
                                     idx_v.at[p, 0], semis[p])
                    pltpu.async_copy(ei_ref.at[1, b + 2 * W],
                                     idx_v.at[p, 1], semid[p])
        return 0
    lax.fori_loop(0, (_NB_AGG + 1) // 2, ebody, 0)
    plsc.subcore_barrier()
    pltpu.sync_copy(accum.at[pl.ds(s * _RPT, _RPT)],
                    out_ref.at[c, pl.ds(s * _RPT, _RPT)])


def _sc_agg(rows, ei):
    return pl.kernel(
        _sc_agg_body,
        out_type=jax.ShapeDtypeStruct((2, NP, F), jnp.float32),
        mesh=plsc.VectorSubcoreMesh(**_MESH),
        compiler_params=_SC_PARAMS,
        scratch_types=[
            pltpu.VMEM((2, 2, 128), jnp.int32),
            pltpu.VMEM((2, 128, F), jnp.float32),
            pltpu.VMEM_SHARED((NP, F), jnp.float32),
            pltpu.SemaphoreType.DMA,
            pltpu.SemaphoreType.DMA,
            pltpu.SemaphoreType.DMA,
            pltpu.SemaphoreType.DMA,
            pltpu.SemaphoreType.DMA,
            pltpu.SemaphoreType.DMA,
        ],
    )(rows, ei)


_DSLOT = -(-EBLK // (2 * NS))


def _sc_dots_body(z_ref, pe_ref, out_ref, idx_v, U, V, P, sbuf,
                  semiu0, semiu1, semiv0, semiv1, semu0, semu1, semv0, semv1,
                  semo0, semo1):
    c = lax.axis_index("c")
    s = lax.axis_index("s")
    semiu = (semiu0, semiu1)
    semiv = (semiv0, semiv1)
    semu = (semu0, semu1)
    semv = (semv0, semv1)
    semo = (semo0, semo1)
    iota16 = lax.iota(jnp.int32, 16)

    def start_gathers(p):
        for h in (0, 1):
            pltpu.async_copy(z_ref.at[idx_v.at[p, 0, h]],
                             U.at[p, pl.ds(h * 128, 128)], semu[p])
            pltpu.async_copy(z_ref.at[idx_v.at[p, 1, h]],
                             V.at[p, pl.ds(h * 128, 128)], semv[p])

    def wait_gathers(p):
        for h in (0, 1):
            pltpu.make_async_copy(z_ref.at[idx_v.at[p, 0, h]],
                                  U.at[p, pl.ds(h * 128, 128)],
                                  semu[p]).wait()
            pltpu.make_async_copy(z_ref.at[idx_v.at[p, 1, h]],
                                  V.at[p, pl.ds(h * 128, 128)],
                                  semv[p]).wait()

    b0 = 2 * s
    pltpu.sync_copy(pe_ref.at[c, 0, pl.ds(b0, 2)], idx_v.at[0, 0])
    pltpu.sync_copy(pe_ref.at[c, 1, pl.ds(b0, 2)], idx_v.at[0, 1])
    start_gathers(0)
    pltpu.async_copy(pe_ref.at[c, 0, pl.ds(b0 + 2 * NS, 2)], idx_v.at[1, 0],
                     semiu[1])
    pltpu.async_copy(pe_ref.at[c, 1, pl.ds(b0 + 2 * NS, 2)], idx_v.at[1, 1],
                     semiv[1])

    def ebody(i, _):
        for p in (0, 1):
            pn = 1 - p
            t = 2 * i + p
            b = 2 * s + 2 * NS * t

            @pl.when(b < EBLK)
            def _():
                wait_gathers(p)

                @pl.when(b + 4 * NS < EBLK)
                def _():
                    pltpu.async_copy(pe_ref.at[c, 0, pl.ds(b + 4 * NS, 2)],
                                     idx_v.at[p, 0], semiu[p])
                    pltpu.async_copy(pe_ref.at[c, 1, pl.ds(b + 4 * NS, 2)],
                                     idx_v.at[p, 1], semiv[p])

                @pl.when(b + 2 * NS < EBLK)
                def _():
                    pltpu.make_async_copy(
                        pe_ref.at[c, 0, pl.ds(b + 2 * NS, 2)],
                        idx_v.at[pn, 0], semiu[pn]).wait()
                    pltpu.make_async_copy(
                        pe_ref.at[c, 1, pl.ds(b + 2 * NS, 2)],
                        idx_v.at[pn, 1], semiv[pn]).wait()
                    start_gathers(pn)

                @pl.when(i >= 1)
                def _():
                    pltpu.make_async_copy(sbuf.at[p],
                                          out_ref.at[c, pl.ds(b, 2)],
                                          semo[p]).wait()

                def grp(gi, _):
                    for e in range(16):
                        j = gi * 16 + e
                        acc = None
                        for k in range(4):
                            hu = U[p, j, pl.ds(k * 16, 16)]
                            hv = V[p, j, pl.ds(k * 16, 16)]
                            lu = plsc.bitcast(
                                plsc.bitcast(hu, jnp.uint32) << 16,
                                jnp.float32)
                            lv = plsc.bitcast(
                                plsc.bitcast(hv, jnp.uint32) << 16,
                                jnp.float32)
                            term = hu * hv + lu * lv
                            acc = term if acc is None else acc + term
                        plsc.store_scatter(
                            P, [iota16, jnp.full((16,), e, jnp.int32)], acc)
                    sv = P[0, :]
                    for r in range(1, 16):
                        sv = sv + P[r, :]
                    sbuf[p, gi // 8, pl.ds((gi % 8) * 16, 16)] = sv
                    return 0
                lax.fori_loop(0, 16, grp, 0)
                pltpu.async_copy(sbuf.at[p], out_ref.at[c, pl.ds(b, 2)],
                                 semo[p])
        return 0
    lax.fori_loop(0, (_DSLOT + 1) // 2, ebody, 0)
    for p in (0, 1):
        pltpu.make_async_copy(sbuf.at[p], out_ref.at[c, pl.ds(2 * s, 2)],
                              semo[p]).wait()


def _sc_dots(zp, pe4):
    return pl.kernel(
        _sc_dots_body,
        out_type=jax.ShapeDtypeStruct((2, EBLK, 128), jnp.float32),
        mesh=plsc.VectorSubcoreMesh(**_MESH),
        compiler_params=_SC_PARAMS_NT,
        scratch_types=[
            pltpu.VMEM((2, 2, 2, 128), jnp.int32),
            pltpu.VMEM((2, 256, F // 2), jnp.float32),
            pltpu.VMEM((2, 256, F // 2), jnp.float32),
            pltpu.VMEM((16, 16), jnp.float32),
            pltpu.VMEM((2, 2, 128), jnp.float32),
            pltpu.SemaphoreType.DMA,
            pltpu.SemaphoreType.DMA,
            pltpu.SemaphoreType.DMA,
            pltpu.SemaphoreType.DMA,
            pltpu.SemaphoreType.DMA,
            pltpu.SemaphoreType.DMA,
            pltpu.SemaphoreType.DMA,
            pltpu.SemaphoreType.DMA,
            pltpu.SemaphoreType.DMA,
            pltpu.SemaphoreType.DMA,
        ],
    )(zp, pe4)


def kernel(x, block0_edge_index, block1_edge_index, pos_edge_index,
           neg_edge_index, W0, b0, Wm, bm, Ws, bs):
    xp = jnp.pad(x, ((0, NP - N), (0, 0)))

    ei4 = jnp.stack([block0_edge_index, block1_edge_index]).reshape(
        2, 2, EBLK, 128)
    degs2 = _sc_degrees(ei4)
    degs4 = degs2.reshape(4, NP).transpose(1, 0)

    xs, nrm = _tc_norms(degs4, xp)

    p2 = _sc_agg(xs, block0_edge_index.reshape(2, EBLK, 128))
    hs = _tc_layer1(p2, nrm, W0, b0)

    q2 = _sc_agg(hs, block1_edge_index.reshape(2, EBLK, 128))
    noise = jnp.pad(
        jax.random.normal(jax.random.key(42), (N, F), dtype=jnp.float32),
        ((0, NP - N), (0, 0)))
    zp, zq = _tc_layer2(q2, nrm, Wm, bm, Ws, bs, noise)

    pe4 = jnp.stack([pos_edge_index, neg_edge_index]).reshape(2, 2, EBLK, 128)
    sc2 = _sc_dots(zq, pe4)

    return (sc2[0].reshape(E, 1), sc2[1].reshape(E, 1), zp[:N])


# speedup vs baseline: 2.1212x; 1.0018x over previous
import jax
import jax.numpy as jnp
from jax import lax
from jax.experimental import pallas as pl
from jax.experimental.pallas import tpu as pltpu
from jax.experimental.pallas import tpu_sc as plsc

N = 10000
E = 320000
F = 128
NP = 10240
BLK = 1024
GRID = NP // BLK
EBLK = E // 128
NC = 2
NS = 16
SL = 2 * NP // NS

_MESH = dict(core_axis_name="c", subcore_axis_name="s", num_cores=NC,
             num_subcores=NS)
_SC_PARAMS = pltpu.CompilerParams(needs_layout_passes=False)
_SC_PARAMS_NT = pltpu.CompilerParams(needs_layout_passes=False,
                                     use_tc_tiling_on_sc=False)


def _norms_kernel(deg_ref, x_ref, xs_ref, nrm_ref):
    deg = deg_ref[...]
    nrm = jnp.where(deg > 0.0, lax.rsqrt(jnp.where(deg > 0.0, deg, 1.0)), 0.0)
    nrm_ref[...] = nrm
    xs_ref[...] = x_ref[...] * nrm[:, 0:1]


def _layer1_kernel(p_ref, nrm_ref, w_ref, b_ref, hs_ref):
    s = (p_ref[0] + p_ref[1]) * nrm_ref[:, 1:2]
    h = jnp.maximum(jnp.dot(s, w_ref[...], preferred_element_type=jnp.float32)
                    + b_ref[...], 0.0)
    hs_ref[...] = h * nrm_ref[:, 2:3]


def _layer2_kernel(q_ref, nrm_ref, wm_ref, bm_ref, ws_ref, bs_ref, noise_ref,
                   z_ref, zq_ref):
    t = (q_ref[0] + q_ref[1]) * nrm_ref[:, 3:4]
    mean = jnp.dot(t, wm_ref[...], preferred_element_type=jnp.float32) + bm_ref[...]
    log_std = jnp.dot(t, ws_ref[...], preferred_element_type=jnp.float32) + bs_ref[...]
    z = mean + noise_ref[...] * jnp.exp(log_std)
    z_ref[...] = z
    hi = lax.bitcast_convert_type(z[:, :F // 2].astype(jnp.bfloat16),
                                  jnp.uint16).astype(jnp.uint32) << 16
    lo = lax.bitcast_convert_type(z[:, F // 2:].astype(jnp.bfloat16),
                                  jnp.uint16).astype(jnp.uint32)
    zq_ref[...] = lax.bitcast_convert_type(hi | lo, jnp.float32)


def _tc_norms(degs4, xp):
    return pl.pallas_call(
        _norms_kernel,
        grid=(GRID,),
        in_specs=[
            pl.BlockSpec((BLK, 4), lambda i: (i, 0)),
            pl.BlockSpec((BLK, F), lambda i: (i, 0)),
        ],
        out_specs=[
            pl.BlockSpec((BLK, F), lambda i: (i, 0)),
            pl.BlockSpec((BLK, 4), lambda i: (i, 0)),
        ],
        out_shape=[
            jax.ShapeDtypeStruct((NP, F), jnp.float32),
            jax.ShapeDtypeStruct((NP, 4), jnp.float32),
        ],
    )(degs4, xp)


def _tc_layer1(p2, nrm, W0, b0):
    return pl.pallas_call(
        _layer1_kernel,
        grid=(GRID,),
        in_specs=[
            pl.BlockSpec((2, BLK, F), lambda i: (0, i, 0)),
            pl.BlockSpec((BLK, 4), lambda i: (i, 0)),
            pl.BlockSpec((F, F), lambda i: (0, 0)),
            pl.BlockSpec((1, F), lambda i: (0, 0)),
        ],
        out_specs=pl.BlockSpec((BLK, F), lambda i: (i, 0)),
        out_shape=jax.ShapeDtypeStruct((NP, F), jnp.float32),
    )(p2, nrm, W0, b0.reshape(1, F))


def _tc_layer2(q2, nrm, Wm, bm, Ws, bs, noise):
    return pl.pallas_call(
        _layer2_kernel,
        grid=(GRID,),
        in_specs=[
            pl.BlockSpec((2, BLK, F), lambda i: (0, i, 0)),
            pl.BlockSpec((BLK, 4), lambda i: (i, 0)),
            pl.BlockSpec((F, F), lambda i: (0, 0)),
            pl.BlockSpec((1, F), lambda i: (0, 0)),
            pl.BlockSpec((F, F), lambda i: (0, 0)),
            pl.BlockSpec((1, F), lambda i: (0, 0)),
            pl.BlockSpec((BLK, F), lambda i: (i, 0)),
        ],
        out_specs=[
            pl.BlockSpec((BLK, F), lambda i: (i, 0)),
            pl.BlockSpec((BLK, F // 2), lambda i: (i, 0)),
        ],
        out_shape=[
            jax.ShapeDtypeStruct((NP, F), jnp.float32),
            jax.ShapeDtypeStruct((NP, F // 2), jnp.float32),
        ],
    )(q2, nrm, Wm, bm.reshape(1, F), Ws, bs.reshape(1, F), noise)


_NB_AGG = -(-EBLK // (NC * NS))


def _sc_degrees_body(ei_ref, out_ref, idx_v, hist_v, acc_v, tmp_v, shared,
                     semu0, semu1, semv0, semv1):
    c = lax.axis_index("c")
    s = lax.axis_index("s")
    semu = (semu0, semu1)
    semv = (semv0, semv1)
    zeros16 = jnp.zeros((16,), jnp.float32)
    ones16 = jnp.ones((16,), jnp.float32)

    def zbody(i, _):
        hist_v[pl.ds(i * 16, 16)] = zeros16
        return 0
    lax.fori_loop(0, (2 * NP) // 16, zbody, 0)

    pltpu.sync_copy(ei_ref.at[c, 0, pl.ds(2 * s, 2)], idx_v.at[0, 0])
    pltpu.sync_copy(ei_ref.at[c, 1, pl.ds(2 * s, 2)], idx_v.at[0, 1])
    pltpu.async_copy(ei_ref.at[c, 0, pl.ds(2 * s + 2 * NS, 2)],
                     idx_v.at[1, 0], semu[1])
    pltpu.async_copy(ei_ref.at[c, 1, pl.ds(2 * s + 2 * NS, 2)],
                     idx_v.at[1, 1], semv[1])

    def ebody(i, _):
        for p in (0, 1):
            t = 2 * i + p
            b = 2 * s + 2 * NS * t

            @pl.when(b < EBLK)
            def _():
                if p == 0:
                    @pl.when(i >= 1)
                    def _():
                        pltpu.make_async_copy(ei_ref.at[c, 0, pl.ds(b, 2)],
                                              idx_v.at[p, 0], semu[p]).wait()
                        pltpu.make_async_copy(ei_ref.at[c, 1, pl.ds(b, 2)],
                                              idx_v.at[p, 1], semv[p]).wait()
                else:
                    pltpu.make_async_copy(ei_ref.at[c, 0, pl.ds(b, 2)],
                                          idx_v.at[p, 0], semu[p]).wait()
                    pltpu.make_async_copy(ei_ref.at[c, 1, pl.ds(b, 2)],
                                          idx_v.at[p, 1], semv[p]).wait()
                for j in range(16):
                    src16 = idx_v[p, 0, j // 8, pl.ds((j % 8) * 16, 16)]
                    dst16 = idx_v[p, 1, j // 8, pl.ds((j % 8) * 16, 16)]
                    plsc.addupdate_scatter(hist_v, [src16], ones16)
                    plsc.addupdate_scatter(hist_v, [dst16 + NP], ones16)

                @pl.when(b + 4 * NS < EBLK)
                def _():
                    pltpu.async_copy(ei_ref.at[c, 0, pl.ds(b + 4 * NS, 2)],
                                     idx_v.at[p, 0], semu[p])
                    pltpu.async_copy(ei_ref.at[c, 1, pl.ds(b + 4 * NS, 2)],
                                     idx_v.at[p, 1], semv[p])
        return 0
    lax.fori_loop(0, (-(-EBLK // (2 * NS)) + 1) // 2, ebody, 0)

    pltpu.sync_copy(hist_v, shared.at[s])
    plsc.subcore_barrier()
    pltpu.sync_copy(shared.at[0, pl.ds(s * SL, SL)], acc_v)

    def rbody(p, _):
        pltpu.sync_copy(shared.at[p, pl.ds(s * SL, SL)], tmp_v)

        def abody(k, _):
            acc_v[pl.ds(k * 16, 16)] = (acc_v[pl.ds(k * 16, 16)]
                                        + tmp_v[pl.ds(k * 16, 16)])
            return 0
        lax.fori_loop(0, SL // 16, abody, 0)
        return 0
    lax.fori_loop(1, NS, rbody, 0)
    pltpu.sync_copy(acc_v, out_ref.at[c, pl.ds(s * SL, SL)])


def _sc_degrees(ei4):
    return pl.kernel(
        _sc_degrees_body,
        out_type=jax.ShapeDtypeStruct((2, 2 * NP), jnp.float32),
        mesh=plsc.VectorSubcoreMesh(**_MESH),
        compiler_params=_SC_PARAMS,
        scratch_types=[
            pltpu.VMEM((2, 2, 2, 128), jnp.int32),
            pltpu.VMEM((2 * NP,), jnp.float32),
            pltpu.VMEM((SL,), jnp.float32),
            pltpu.VMEM((SL,), jnp.float32),
            pltpu.VMEM_SHARED((NS, 2 * NP), jnp.float32),
            pltpu.SemaphoreType.DMA,
            pltpu.SemaphoreType.DMA,
            pltpu.SemaphoreType.DMA,
            pltpu.SemaphoreType.DMA,
        ],
    )(ei4)


_RPT = NP // NS


def _sc_agg_body(rows_ref, ei_ref, out_ref, idx_v, rowbuf, accum,
                 semis0, semis1, semid0, semid1, semg0, semg1):
    c = lax.axis_index("c")
    s = lax.axis_index("s")
    wid = s * NC + c
    semis = (semis0, semis1)
    semid = (semid0, semid1)
    semg = (semg0, semg1)
    zeros16 = jnp.zeros((16,), jnp.float32)

    def zb(i, _):
        for j in range(8):
            rowbuf[0, i, pl.ds(j * 16, 16)] = zeros16
        return 0
    lax.fori_loop(0, 128, zb, 0)

    def za(k, _):
        pltpu.sync_copy(rowbuf.at[0],
                        accum.at[pl.ds(s * _RPT + k * 128, 128)])
        return 0
    lax.fori_loop(0, _RPT // 128, za, 0)
    plsc.subcore_barrier()

    W = NC * NS
    pltpu.sync_copy(ei_ref.at[0, wid], idx_v.at[0, 0])
    pltpu.sync_copy(ei_ref.at[1, wid], idx_v.at[0, 1])
    pltpu.async_copy(rows_ref.at[idx_v.at[0, 0]], rowbuf.at[0], semg[0])
    pltpu.async_copy(ei_ref.at[0, wid + W], idx_v.at[1, 0], semis[1])
    pltpu.async_copy(ei_ref.at[1, wid + W], idx_v.at[1, 1], semid[1])

    def ebody(i, _):
        for p in (0, 1):
            pn = 1 - p
            t = 2 * i + p
            b = wid + W * t

            @pl.when(b < EBLK)
            def _():
                @pl.when(b + W < EBLK)
                def _():
                    pltpu.make_async_copy(ei_ref.at[0, b + W],
                                          idx_v.at[pn, 0], semis[pn]).wait()
                    pltpu.make_async_copy(ei_ref.at[1, b + W],
                                          idx_v.at[pn, 1], semid[pn]).wait()
                    pltpu.async_copy(rows_ref.at[idx_v.at[pn, 0]],
                                     rowbuf.at[pn], semg[pn])
                pltpu.make_async_copy(rows_ref.at[idx_v.at[p, 0]],
                                      rowbuf.at[p], semg[p]).wait()
                pltpu.sync_copy(rowbuf.at[p], accum.at[idx_v.at[p, 1]],
                                add=True)

                @pl.when(b + 2 * W < EBLK)
                def _():
                    pltpu.async_copy(ei_ref.at[0, b + 2 * W],
                                     idx_v.at[p, 0], semis[p])
                    pltpu.async_copy(ei_ref.at[1, b + 2 * W],
                                     idx_v.at[p, 1], semid[p])
        return 0
    lax.fori_loop(0, (_NB_AGG + 1) // 2, ebody, 0)
    plsc.subcore_barrier()
    pltpu.sync_copy(accum.at[pl.ds(s * _RPT, _RPT)],
                    out_ref.at[c, pl.ds(s * _RPT, _RPT)])


def _sc_agg(rows, ei):
    return pl.kernel(
        _sc_agg_body,
        out_type=jax.ShapeDtypeStruct((2, NP, F), jnp.float32),
        mesh=plsc.VectorSubcoreMesh(**_MESH),
        compiler_params=_SC_PARAMS,
        scratch_types=[
            pltpu.VMEM((2, 2, 128), jnp.int32),
            pltpu.VMEM((2, 128, F), jnp.float32),
            pltpu.VMEM_SHARED((NP, F), jnp.float32),
            pltpu.SemaphoreType.DMA,
            pltpu.SemaphoreType.DMA,
            pltpu.SemaphoreType.DMA,
            pltpu.SemaphoreType.DMA,
            pltpu.SemaphoreType.DMA,
            pltpu.SemaphoreType.DMA,
        ],
    )(rows, ei)


_DSLOT = -(-EBLK // (2 * NS))


def _sc_dots_body(z_ref, pe_ref, out_ref, idx_v, U, V, P, sbuf,
                  semiu0, semiu1, semiv0, semiv1, semu0, semu1, semv0, semv1,
                  semo0, semo1):
    c = lax.axis_index("c")
    s = lax.axis_index("s")
    semiu = (semiu0, semiu1)
    semiv = (semiv0, semiv1)
    semu = (semu0, semu1)
    semv = (semv0, semv1)
    semo = (semo0, semo1)
    iota16 = lax.iota(jnp.int32, 16)

    def start_gathers(p):
        for h in (0, 1):
            pltpu.async_copy(z_ref.at[idx_v.at[p, 0, h]],
                             U.at[p, pl.ds(h * 128, 128)], semu[p])
            pltpu.async_copy(z_ref.at[idx_v.at[p, 1, h]],
                             V.at[p, pl.ds(h * 128, 128)], semv[p])

    def wait_gathers(p):
        for h in (0, 1):
            pltpu.make_async_copy(z_ref.at[idx_v.at[p, 0, h]],
                                  U.at[p, pl.ds(h * 128, 128)],
                                  semu[p]).wait()
            pltpu.make_async_copy(z_ref.at[idx_v.at[p, 1, h]],
                                  V.at[p, pl.ds(h * 128, 128)],
                                  semv[p]).wait()

    b0 = 2 * s
    pltpu.sync_copy(pe_ref.at[c, 0, pl.ds(b0, 2)], idx_v.at[0, 0])
    pltpu.sync_copy(pe_ref.at[c, 1, pl.ds(b0, 2)], idx_v.at[0, 1])
    start_gathers(0)
    pltpu.async_copy(pe_ref.at[c, 0, pl.ds(b0 + 2 * NS, 2)], idx_v.at[1, 0],
                     semiu[1])
    pltpu.async_copy(pe_ref.at[c, 1, pl.ds(b0 + 2 * NS, 2)], idx_v.at[1, 1],
                     semiv[1])

    def ebody(i, _):
        for p in (0, 1):
            pn = 1 - p
            t = 2 * i + p
            b = 2 * s + 2 * NS * t

            @pl.when(b < EBLK)
            def _():
                wait_gathers(p)

                @pl.when(b + 4 * NS < EBLK)
                def _():
                    pltpu.async_copy(pe_ref.at[c, 0, pl.ds(b + 4 * NS, 2)],
                                     idx_v.at[p, 0], semiu[p])
                    pltpu.async_copy(pe_ref.at[c, 1, pl.ds(b + 4 * NS, 2)],
                                     idx_v.at[p, 1], semiv[p])

                @pl.when(b + 2 * NS < EBLK)
                def _():
                    pltpu.make_async_copy(
                        pe_ref.at[c, 0, pl.ds(b + 2 * NS, 2)],
                        idx_v.at[pn, 0], semiu[pn]).wait()
                    pltpu.make_async_copy(
                        pe_ref.at[c, 1, pl.ds(b + 2 * NS, 2)],
                        idx_v.at[pn, 1], semiv[pn]).wait()
                    start_gathers(pn)

                @pl.when(i >= 1)
                def _():
                    pltpu.make_async_copy(sbuf.at[p],
                                          out_ref.at[c, pl.ds(b, 2)],
                                          semo[p]).wait()

                def grp(gi, _):
                    for e in range(16):
                        j = gi * 16 + e
                        acc = None
                        for k in range(4):
                            hu = U[p, j, pl.ds(k * 16, 16)]
                            hv = V[p, j, pl.ds(k * 16, 16)]
                            lu = plsc.bitcast(
                                plsc.bitcast(hu, jnp.uint32) << 16,
                                jnp.float32)
                            lv = plsc.bitcast(
                                plsc.bitcast(hv, jnp.uint32) << 16,
                                jnp.float32)
                            term = hu * hv + lu * lv
                            acc = term if acc is None else acc + term
                        plsc.store_scatter(
                            P, [iota16, jnp.full((16,), e, jnp.int32)], acc)
                    sv = P[0, :]
                    for r in range(1, 16):
                        sv = sv + P[r, :]
                    sbuf[p, gi // 8, pl.ds((gi % 8) * 16, 16)] = sv
                    return 0
                lax.fori_loop(0, 16, grp, 0)
                pltpu.async_copy(sbuf.at[p], out_ref.at[c, pl.ds(b, 2)],
                                 semo[p])
        return 0
    lax.fori_loop(0, (_DSLOT + 1) // 2, ebody, 0)
    for p in (0, 1):
        pltpu.make_async_copy(sbuf.at[p], out_ref.at[c, pl.ds(2 * s, 2)],
                              semo[p]).wait()


def _sc_dots(zp, pe4):
    return pl.kernel(
        _sc_dots_body,
        out_type=jax.ShapeDtypeStruct((2, EBLK, 128), jnp.float32),
        mesh=plsc.VectorSubcoreMesh(**_MESH),
        compiler_params=_SC_PARAMS_NT,
        scratch_types=[
            pltpu.VMEM((2, 2, 2, 128), jnp.int32),
            pltpu.VMEM((2, 256, F // 2), jnp.float32),
            pltpu.VMEM((2, 256, F // 2), jnp.float32),
            pltpu.VMEM((16, 16), jnp.float32),
            pltpu.VMEM((2, 2, 128), jnp.float32),
            pltpu.SemaphoreType.DMA,
            pltpu.SemaphoreType.DMA,
            pltpu.SemaphoreType.DMA,
            pltpu.SemaphoreType.DMA,
            pltpu.SemaphoreType.DMA,
            pltpu.SemaphoreType.DMA,
            pltpu.SemaphoreType.DMA,
            pltpu.SemaphoreType.DMA,
            pltpu.SemaphoreType.DMA,
            pltpu.SemaphoreType.DMA,
        ],
    )(zp, pe4)


def kernel(x, block0_edge_index, block1_edge_index, pos_edge_index,
           neg_edge_index, W0, b0, Wm, bm, Ws, bs):
    xp = jnp.pad(x, ((0, NP - N), (0, 0)))

    ei4 = jnp.stack([block0_edge_index, block1_edge_index]).reshape(
        2, 2, EBLK, 128)
    degs2 = _sc_degrees(ei4)
    degs4 = degs2.reshape(4, NP).transpose(1, 0)

    xs, nrm = _tc_norms(degs4, xp)

    p2 = _sc_agg(xs, block0_edge_index.reshape(2, EBLK, 128))
    hs = _tc_layer1(p2, nrm, W0, b0)

    q2 = _sc_agg(hs, block1_edge_index.reshape(2, EBLK, 128))
    noise = jnp.pad(
        jax.random.normal(jax.random.key(42), (N, F), dtype=jnp.float32),
        ((0, NP - N), (0, 0)))
    zp, zq = _tc_layer2(q2, nrm, Wm, bm, Ws, bs, noise)

    pe4 = jnp.stack([pos_edge_index, neg_edge_index]).reshape(2, 2, EBLK, 128)
    sc2 = _sc_dots(zq, pe4)

    return (sc2[0].reshape(E, 1), sc2[1].reshape(E, 1), zp[:N])
